# Initial kernel scaffold; baseline (speedup 1.0000x reference)
import functools

import jax
import jax.numpy as jnp
import numpy as np
from jax import lax
from jax.experimental import pallas as pl
from jax.experimental.pallas import tpu as pltpu
from jax.experimental.pallas import tpu_sc as plsc

N = 10000
E = 160000
PD = 256
HD = 512
NUM_OBJ = 151
ORT_DIMS = 200
RELP = 64
NP_ = 10240

NC, NS = 2, 16
NW = NC * NS
EPW = E // NW
CH = 40
NCHUNK = EPW // CH

BLKN = 1024
BLKE = 2000

_F32 = jnp.float32

_sc_mesh = plsc.VectorSubcoreMesh(
    core_axis_name="c", subcore_axis_name="s", num_cores=NC, num_subcores=NS)


def _wid():
    return lax.axis_index("s") * NC + lax.axis_index("c")


def _node_body(ofeat, octx, posp, lab, ortp,
               W_low, b_low, W_high, b_high, W_s, b_s, W_o, b_o, W_ws, b_ws,
               W_wo, sub_o, obj_o, hs_o, ho_o):
    wl = W_low[...]
    label_t = jnp.dot(ortp[...], wl[261:461, :],
                      preferred_element_type=_F32)
    iot = lax.broadcasted_iota(jnp.int32, (BLKN, 256), 1)
    oh = (iot == lab[...]).astype(_F32)
    rep = (jnp.dot(ofeat[...], wl[:256, :], preferred_element_type=_F32)
           + jnp.dot(posp[...], wl[256:264, :], preferred_element_type=_F32)
           + jnp.dot(oh, label_t, preferred_element_type=_F32)
           + b_low[...]
           + jnp.dot(octx[...], W_high[...], preferred_element_type=_F32)
           + b_high[...])
    sub = jnp.dot(rep, W_s[...], preferred_element_type=_F32) + b_s[...]
    obj = jnp.dot(rep, W_o[...], preferred_element_type=_F32) + b_o[...]
    sub_o[...] = sub
    obj_o[...] = obj
    hs_o[...] = jnp.dot(sub, W_ws[...], preferred_element_type=_F32)
    ho_o[...] = jnp.dot(obj, W_wo[...], preferred_element_type=_F32) + b_ws[...]


def _node_stage(ofeat, octx, posp, lab, ortp, W_low, b_low, W_high, b_high,
                W_s, b_s, W_o, b_o, W_ws, b_ws, W_wo):
    grid = NP_ // BLKN
    row = lambda i: (i, 0)
    fixed = lambda i: (0, 0)
    return pl.pallas_call(
        _node_body,
        grid=(grid,),
        in_specs=[
            pl.BlockSpec((BLKN, PD), row),
            pl.BlockSpec((BLKN, HD), row),
            pl.BlockSpec((BLKN, 8), row),
            pl.BlockSpec((BLKN, 1), row),
            pl.BlockSpec((256, ORT_DIMS), fixed),
            pl.BlockSpec((461, PD), fixed),
            pl.BlockSpec((1, PD), fixed),
            pl.BlockSpec((HD, PD), fixed),
            pl.BlockSpec((1, PD), fixed),
            pl.BlockSpec((PD, PD), fixed),
            pl.BlockSpec((1, PD), fixed),
            pl.BlockSpec((PD, PD), fixed),
            pl.BlockSpec((1, PD), fixed),
            pl.BlockSpec((PD, HD), fixed),
            pl.BlockSpec((1, HD), fixed),
            pl.BlockSpec((PD, HD), fixed),
        ],
        out_specs=[
            pl.BlockSpec((BLKN, PD), row),
            pl.BlockSpec((BLKN, PD), row),
            pl.BlockSpec((BLKN, HD), row),
            pl.BlockSpec((BLKN, HD), row),
        ],
        out_shape=[
            jax.ShapeDtypeStruct((NP_, PD), _F32),
            jax.ShapeDtypeStruct((NP_, PD), _F32),
            jax.ShapeDtypeStruct((NP_, HD), _F32),
            jax.ShapeDtypeStruct((NP_, HD), _F32),
        ],
    )(ofeat, octx, posp, lab, ortp, W_low, b_low, W_high, b_high,
      W_s, b_s, W_o, b_o, W_ws, b_ws, W_wo)


def _gather2_body(hs, ho, src, dst, g1, g2, idx_s, idx_d, buf1, buf2, sem):
    base = _wid() * EPW

    def step(i, _):
        off = base + i * CH
        pltpu.sync_copy(src.at[pl.ds(off, CH)], idx_s)
        pltpu.sync_copy(dst.at[pl.ds(off, CH)], idx_d)
        c1 = pltpu.async_copy(hs.at[idx_s], buf1, sem)
        c2 = pltpu.async_copy(ho.at[idx_d], buf2, sem)
        c1.wait()
        c2.wait()
        pltpu.sync_copy(buf1, g1.at[pl.ds(off, CH)])
        pltpu.sync_copy(buf2, g2.at[pl.ds(off, CH)])
        return 0

    lax.fori_loop(0, NCHUNK, step, 0)


_gather2 = functools.partial(
    pl.kernel,
    out_type=[jax.ShapeDtypeStruct((E, HD), _F32),
              jax.ShapeDtypeStruct((E, HD), _F32)],
    mesh=_sc_mesh,
    scratch_types=[
        pltpu.VMEM((CH,), jnp.int32),
        pltpu.VMEM((CH,), jnp.int32),
        pltpu.VMEM((CH, HD), _F32),
        pltpu.VMEM((CH, HD), _F32),
        pltpu.SemaphoreType.DMA,
    ],
)(_gather2_body)


def _edge1_body(u, g1, g2, W_wu, w_hmp, b_hmp, Wr1p, mw_o):
    uu = u[...]
    t = jnp.dot(uu, W_wu[...], preferred_element_type=_F32)
    h = jnp.tanh(g1[...] + g2[...] + t)
    logit = jnp.sum(h * w_hmp[...], axis=1, keepdims=True) + b_hmp[...]
    att = jax.nn.sigmoid(logit)
    mw_o[...] = att * jnp.dot(uu, Wr1p[...], preferred_element_type=_F32)


def _edge1(u, g1, g2, W_wu, w_hmp, b_hmp, Wr1p):
    grid = E // BLKE
    row = lambda i: (i, 0)
    fixed = lambda i: (0, 0)
    return pl.pallas_call(
        _edge1_body,
        grid=(grid,),
        in_specs=[
            pl.BlockSpec((BLKE, PD), row),
            pl.BlockSpec((BLKE, HD), row),
            pl.BlockSpec((BLKE, HD), row),
            pl.BlockSpec((PD, HD), fixed),
            pl.BlockSpec((1, HD), fixed),
            pl.BlockSpec((1, 1), fixed),
            pl.BlockSpec((PD, RELP), fixed),
        ],
        out_specs=pl.BlockSpec((BLKE, RELP), row),
        out_shape=jax.ShapeDtypeStruct((E, RELP), _F32),
    )(u, g1, g2, W_wu, w_hmp, b_hmp, Wr1p)


def _scatter_body(mw, src, zer, out, idx_v, buf, aggsh):
    cid = lax.axis_index("c")
    sid = lax.axis_index("s")
    wid = sid * NC + cid
    rps = NP_ // NS
    pltpu.sync_copy(zer.at[pl.ds(sid * rps, rps)],
                    aggsh.at[pl.ds(sid * rps, rps)])
    plsc.subcore_barrier()
    base = wid * EPW

    def step(i, _):
        off = base + i * CH
        pltpu.sync_copy(src.at[pl.ds(off, CH)], idx_v)
        pltpu.sync_copy(mw.at[pl.ds(off, CH)], buf)
        pltpu.sync_copy(buf, aggsh.at[idx_v], add=True)
        return 0

    lax.fori_loop(0, NCHUNK, step, 0)
    plsc.subcore_barrier()
    pltpu.sync_copy(aggsh.at[pl.ds(sid * rps, rps)],
                    out.at[pl.ds(cid * NP_ + sid * rps, rps)])


_scatter = functools.partial(
    pl.kernel,
    out_type=jax.ShapeDtypeStruct((2 * NP_, RELP), _F32),
    mesh=_sc_mesh,
    scratch_types=[
        pltpu.VMEM((CH,), jnp.int32),
        pltpu.VMEM((CH, RELP), _F32),
        pltpu.VMEM_SHARED((NP_, RELP), _F32),
    ],
)(_scatter_body)


def _combine_body(p0, p1, out_o):
    out_o[...] = p0[...] + p1[...]


def _combine(parts):
    grid = NP_ // BLKN
    row = lambda i: (i, 0)
    return pl.pallas_call(
        _combine_body,
        grid=(grid,),
        in_specs=[pl.BlockSpec((BLKN, RELP), row),
                  pl.BlockSpec((BLKN, RELP), row)],
        out_specs=pl.BlockSpec((BLKN, RELP), row),
        out_shape=jax.ShapeDtypeStruct((NP_, RELP), _F32),
    )(parts[:NP_], parts[NP_:])


def _gather3_body(sub, obj, aggw, src, dst, s1, s2, ag,
                  idx_s, idx_d, b1, b2, b3, sem):
    base = _wid() * EPW

    def step(i, _):
        off = base + i * CH
        pltpu.sync_copy(src.at[pl.ds(off, CH)], idx_s)
        pltpu.sync_copy(dst.at[pl.ds(off, CH)], idx_d)
        c1 = pltpu.async_copy(sub.at[idx_s], b1, sem)
        c2 = pltpu.async_copy(obj.at[idx_d], b2, sem)
        c3 = pltpu.async_copy(aggw.at[idx_s], b3, sem)
        c1.wait()
        c2.wait()
        c3.wait()
        pltpu.sync_copy(b1, s1.at[pl.ds(off, CH)])
        pltpu.sync_copy(b2, s2.at[pl.ds(off, CH)])
        pltpu.sync_copy(b3, ag.at[pl.ds(off, CH)])
        return 0

    lax.fori_loop(0, NCHUNK, step, 0)


_gather3 = functools.partial(
    pl.kernel,
    out_type=[jax.ShapeDtypeStruct((E, PD), _F32),
              jax.ShapeDtypeStruct((E, PD), _F32),
              jax.ShapeDtypeStruct((E, RELP), _F32)],
    mesh=_sc_mesh,
    scratch_types=[
        pltpu.VMEM((CH,), jnp.int32),
        pltpu.VMEM((CH,), jnp.int32),
        pltpu.VMEM((CH, PD), _F32),
        pltpu.VMEM((CH, PD), _F32),
        pltpu.VMEM((CH, RELP), _F32),
        pltpu.SemaphoreType.DMA,
    ],
)(_gather3_body)


def _edge2_body(s1, s2, ag, mw, spt, Wr1p, Wr2p, brel, out_o):
    p = s1[...] * s2[...]
    out_o[...] = (jnp.dot(p, Wr1p[...], preferred_element_type=_F32)
                  + jnp.dot(spt[...], Wr2p[...], preferred_element_type=_F32)
                  + ag[...] + mw[...] + brel[...])


def _edge2(s1, s2, ag, mw, spt, Wr1p, Wr2p, brel):
    grid = E // BLKE
    row = lambda i: (i, 0)
    fixed = lambda i: (0, 0)
    return pl.pallas_call(
        _edge2_body,
        grid=(grid,),
        in_specs=[
            pl.BlockSpec((BLKE, PD), row),
            pl.BlockSpec((BLKE, PD), row),
            pl.BlockSpec((BLKE, RELP), row),
            pl.BlockSpec((BLKE, RELP), row),
            pl.BlockSpec((BLKE, 64), row),
            pl.BlockSpec((PD, RELP), fixed),
            pl.BlockSpec((64, RELP), fixed),
            pl.BlockSpec((1, RELP), fixed),
        ],
        out_specs=pl.BlockSpec((BLKE, RELP), row),
        out_shape=jax.ShapeDtypeStruct((E, RELP), _F32),
    )(s1, s2, ag, mw, spt, Wr1p, Wr2p, brel)


def kernel(obj_feats, obj_ctx, pos_embed, union_feats, spt_feats, pair_idx,
           obj_labels, W_low, b_low, W_high, b_high, W_s, b_s, W_o, b_o,
           W_ws, b_ws, W_wo, W_wu, W_hmp, b_hmp, W_rel, b_rel):
    ind = jnp.arange(1, NUM_OBJ + 1, dtype=_F32)[:, None]
    lin = jnp.linspace(-np.pi, np.pi, ORT_DIMS, dtype=_F32)[None, :]
    t = ind * lin
    ortp = jnp.zeros((256, ORT_DIMS), _F32).at[:NUM_OBJ].set(
        jnp.sin(t) + jnp.cos(t))

    pad_n = NP_ - N
    ofeat = jnp.pad(obj_feats, ((0, pad_n), (0, 0)))
    octx = jnp.pad(obj_ctx, ((0, pad_n), (0, 0)))
    posp = jnp.pad(pos_embed, ((0, pad_n), (0, 3)))
    lab = jnp.pad(obj_labels.astype(jnp.int32), (0, pad_n))[:, None]

    src = pair_idx[:, 0].astype(jnp.int32)
    dst = pair_idx[:, 1].astype(jnp.int32)

    Wr1p = jnp.pad(W_rel[:PD], ((0, 0), (0, RELP - 51)))
    Wr2p = jnp.pad(W_rel[PD:], ((0, 0), (0, RELP - 51)))
    brelp = jnp.pad(b_rel, (0, RELP - 51))[None, :]
    w_hmp = W_hmp.reshape(1, HD)
    b_hmp2 = b_hmp.reshape(1, 1)
    zer = jnp.zeros((NP_, RELP), _F32)

    sub, obj, hs, ho = _node_stage(
        ofeat, octx, posp, lab, ortp, W_low, b_low[None, :], W_high,
        b_high[None, :], W_s, b_s[None, :], W_o, b_o[None, :], W_ws,
        b_ws[None, :], W_wo)

    g1, g2 = _gather2(hs, ho, src, dst)

    mw = _edge1(union_feats, g1, g2, W_wu, w_hmp, b_hmp2, Wr1p)

    parts = _scatter(mw, src, zer)
    aggw = _combine(parts)

    s1, s2, ag = _gather3(sub, obj, aggw, src, dst)

    out64 = _edge2(s1, s2, ag, mw, spt_feats, Wr1p, Wr2p, brelp)
    return out64[:, :51]


# speedup vs baseline: 1.4466x; 1.4466x over previous
import functools

import jax
import jax.numpy as jnp
import numpy as np
from jax import lax
from jax.experimental import pallas as pl
from jax.experimental.pallas import tpu as pltpu
from jax.experimental.pallas import tpu_sc as plsc

N = 10000
E = 160000
PD = 256
HD = 512
NUM_OBJ = 151
ORT_DIMS = 200
RELP = 64
RELW = 128
SRCW = PD + RELW
NP_ = 10240

NC, NS = 2, 16
NW = NC * NS
EPW = E // NW
CH = 40
NCHUNK = EPW // CH

BLKN = 1024
BLKE = 2000

_F32 = jnp.float32

_sc_mesh = plsc.VectorSubcoreMesh(
    core_axis_name="c", subcore_axis_name="s", num_cores=NC, num_subcores=NS)


def _wid():
    return lax.axis_index("s") * NC + lax.axis_index("c")


def _node_body(ofeat, octx, posp, lab, ortp,
               W_low, b_low, W_high, b_high, W_s, b_s, W_o, b_o, W_ws, b_ws,
               W_wo, sub_o, obj_o, hs_o, ho_o):
    wl = W_low[...]
    label_t = jnp.dot(ortp[...], wl[261:461, :],
                      preferred_element_type=_F32)
    iot = lax.broadcasted_iota(jnp.int32, (BLKN, 256), 1)
    oh = (iot == lab[...]).astype(_F32)
    rep = (jnp.dot(ofeat[...], wl[:256, :], preferred_element_type=_F32)
           + jnp.dot(posp[...], wl[256:264, :], preferred_element_type=_F32)
           + jnp.dot(oh, label_t, preferred_element_type=_F32)
           + b_low[...]
           + jnp.dot(octx[...], W_high[...], preferred_element_type=_F32)
           + b_high[...])
    sub = jnp.dot(rep, W_s[...], preferred_element_type=_F32) + b_s[...]
    obj = jnp.dot(rep, W_o[...], preferred_element_type=_F32) + b_o[...]
    sub_o[...] = sub
    obj_o[...] = obj
    hs_o[...] = jnp.dot(sub, W_ws[...], preferred_element_type=_F32)
    ho_o[...] = jnp.dot(obj, W_wo[...], preferred_element_type=_F32) + b_ws[...]


def _node_stage(ofeat, octx, posp, lab, ortp, W_low, b_low, W_high, b_high,
                W_s, b_s, W_o, b_o, W_ws, b_ws, W_wo):
    grid = NP_ // BLKN
    row = lambda i: (i, 0)
    fixed = lambda i: (0, 0)
    return pl.pallas_call(
        _node_body,
        grid=(grid,),
        in_specs=[
            pl.BlockSpec((BLKN, PD), row),
            pl.BlockSpec((BLKN, HD), row),
            pl.BlockSpec((BLKN, 8), row),
            pl.BlockSpec((BLKN, 1), row),
            pl.BlockSpec((256, ORT_DIMS), fixed),
            pl.BlockSpec((461, PD), fixed),
            pl.BlockSpec((1, PD), fixed),
            pl.BlockSpec((HD, PD), fixed),
            pl.BlockSpec((1, PD), fixed),
            pl.BlockSpec((PD, PD), fixed),
            pl.BlockSpec((1, PD), fixed),
            pl.BlockSpec((PD, PD), fixed),
            pl.BlockSpec((1, PD), fixed),
            pl.BlockSpec((PD, HD), fixed),
            pl.BlockSpec((1, HD), fixed),
            pl.BlockSpec((PD, HD), fixed),
        ],
        out_specs=[
            pl.BlockSpec((BLKN, PD), row),
            pl.BlockSpec((BLKN, PD), row),
            pl.BlockSpec((BLKN, HD), row),
            pl.BlockSpec((BLKN, HD), row),
        ],
        out_shape=[
            jax.ShapeDtypeStruct((NP_, PD), _F32),
            jax.ShapeDtypeStruct((NP_, PD), _F32),
            jax.ShapeDtypeStruct((NP_, HD), _F32),
            jax.ShapeDtypeStruct((NP_, HD), _F32),
        ],
    )(ofeat, octx, posp, lab, ortp, W_low, b_low, W_high, b_high,
      W_s, b_s, W_o, b_o, W_ws, b_ws, W_wo)


def _gather2_body(hs, ho, src, dst, g1, g2, idx_s, idx_d, buf1, buf2, sem):
    base = _wid() * EPW

    def step(i, _):
        off = base + i * CH
        pltpu.sync_copy(src.at[pl.ds(off, CH)], idx_s)
        pltpu.sync_copy(dst.at[pl.ds(off, CH)], idx_d)
        c1 = pltpu.async_copy(hs.at[idx_s], buf1, sem)
        c2 = pltpu.async_copy(ho.at[idx_d], buf2, sem)
        c1.wait()
        c2.wait()
        pltpu.sync_copy(buf1, g1.at[pl.ds(off, CH)])
        pltpu.sync_copy(buf2, g2.at[pl.ds(off, CH)])
        return 0

    lax.fori_loop(0, NCHUNK, step, 0)


_gather2 = functools.partial(
    pl.kernel,
    out_type=[jax.ShapeDtypeStruct((E, HD), _F32),
              jax.ShapeDtypeStruct((E, HD), _F32)],
    mesh=_sc_mesh,
    scratch_types=[
        pltpu.VMEM((CH,), jnp.int32),
        pltpu.VMEM((CH,), jnp.int32),
        pltpu.VMEM((CH, HD), _F32),
        pltpu.VMEM((CH, HD), _F32),
        pltpu.SemaphoreType.DMA,
    ],
)(_gather2_body)


def _edge1_body(u, g1, g2, W_wu, w_hmp, b_hmp, Wr1p, mw_o):
    uu = u[...]
    t = jnp.dot(uu, W_wu[...], preferred_element_type=_F32)
    h = jnp.tanh(g1[...] + g2[...] + t)
    logit = jnp.sum(h * w_hmp[...], axis=1, keepdims=True) + b_hmp[...]
    att = jax.nn.sigmoid(logit)
    mw_o[...] = att * jnp.dot(uu, Wr1p[...], preferred_element_type=_F32)


def _edge1(u, g1, g2, W_wu, w_hmp, b_hmp, Wr1p):
    grid = E // BLKE
    row = lambda i: (i, 0)
    fixed = lambda i: (0, 0)
    return pl.pallas_call(
        _edge1_body,
        grid=(grid,),
        in_specs=[
            pl.BlockSpec((BLKE, PD), row),
            pl.BlockSpec((BLKE, HD), row),
            pl.BlockSpec((BLKE, HD), row),
            pl.BlockSpec((PD, HD), fixed),
            pl.BlockSpec((1, HD), fixed),
            pl.BlockSpec((1, 1), fixed),
            pl.BlockSpec((PD, RELW), fixed),
        ],
        out_specs=pl.BlockSpec((BLKE, RELW), row),
        out_shape=jax.ShapeDtypeStruct((E, RELW), _F32),
    )(u, g1, g2, W_wu, w_hmp, b_hmp, Wr1p)


def _scatter_body(mw, src, zer, out, idx_v, buf, aggsh):
    cid = lax.axis_index("c")
    sid = lax.axis_index("s")
    wid = sid * NC + cid
    rps = NP_ // NS
    pltpu.sync_copy(zer.at[pl.ds(sid * rps, rps)],
                    aggsh.at[pl.ds(sid * rps, rps)])
    plsc.subcore_barrier()
    base = wid * EPW

    def step(i, _):
        off = base + i * CH
        pltpu.sync_copy(src.at[pl.ds(off, CH)], idx_v)
        pltpu.sync_copy(mw.at[pl.ds(off, CH)], buf)
        pltpu.sync_copy(buf, aggsh.at[idx_v], add=True)
        return 0

    lax.fori_loop(0, NCHUNK, step, 0)
    plsc.subcore_barrier()
    pltpu.sync_copy(aggsh.at[pl.ds(sid * rps, rps)],
                    out.at[pl.ds(cid * NP_ + sid * rps, rps)])


_scatter = functools.partial(
    pl.kernel,
    out_type=jax.ShapeDtypeStruct((2 * NP_, RELW), _F32),
    mesh=_sc_mesh,
    scratch_types=[
        pltpu.VMEM((CH,), jnp.int32),
        pltpu.VMEM((CH, RELW), _F32),
        pltpu.VMEM_SHARED((NP_, RELW), _F32),
    ],
)(_scatter_body)


def _combine_body(p0, p1, sub, out_o):
    out_o[...] = jnp.concatenate([sub[...], p0[...] + p1[...]], axis=1)


def _combine(parts, sub):
    grid = NP_ // BLKN
    row = lambda i: (i, 0)
    return pl.pallas_call(
        _combine_body,
        grid=(grid,),
        in_specs=[pl.BlockSpec((BLKN, RELW), row),
                  pl.BlockSpec((BLKN, RELW), row),
                  pl.BlockSpec((BLKN, PD), row)],
        out_specs=pl.BlockSpec((BLKN, SRCW), row),
        out_shape=jax.ShapeDtypeStruct((NP_, SRCW), _F32),
    )(parts[:NP_], parts[NP_:], sub)


def _gather3_body(tsrc, obj, src, dst, gs, s2, idx_s, idx_d, b1, b2, sem):
    base = _wid() * EPW

    def step(i, _):
        off = base + i * CH
        pltpu.sync_copy(src.at[pl.ds(off, CH)], idx_s)
        pltpu.sync_copy(dst.at[pl.ds(off, CH)], idx_d)
        c1 = pltpu.async_copy(tsrc.at[idx_s], b1, sem)
        c2 = pltpu.async_copy(obj.at[idx_d], b2, sem)
        c1.wait()
        c2.wait()
        pltpu.sync_copy(b1, gs.at[pl.ds(off, CH)])
        pltpu.sync_copy(b2, s2.at[pl.ds(off, CH)])
        return 0

    lax.fori_loop(0, NCHUNK, step, 0)


_gather3 = functools.partial(
    pl.kernel,
    out_type=[jax.ShapeDtypeStruct((E, SRCW), _F32),
              jax.ShapeDtypeStruct((E, PD), _F32)],
    mesh=_sc_mesh,
    scratch_types=[
        pltpu.VMEM((CH,), jnp.int32),
        pltpu.VMEM((CH,), jnp.int32),
        pltpu.VMEM((CH, SRCW), _F32),
        pltpu.VMEM((CH, PD), _F32),
        pltpu.SemaphoreType.DMA,
    ],
)(_gather3_body)


def _edge2_body(gs, s2, mw, spt, Wr1p, Wr2p, brel, out_o):
    g = gs[...]
    p = g[:, :PD] * s2[...]
    out_o[...] = (jnp.dot(p, Wr1p[...], preferred_element_type=_F32)
                  + jnp.dot(spt[...], Wr2p[...], preferred_element_type=_F32)
                  + g[:, PD:PD + RELP] + mw[...][:, :RELP] + brel[...])


def _edge2(gs, s2, mw, spt, Wr1p, Wr2p, brel):
    grid = E // BLKE
    row = lambda i: (i, 0)
    fixed = lambda i: (0, 0)
    return pl.pallas_call(
        _edge2_body,
        grid=(grid,),
        in_specs=[
            pl.BlockSpec((BLKE, SRCW), row),
            pl.BlockSpec((BLKE, PD), row),
            pl.BlockSpec((BLKE, RELW), row),
            pl.BlockSpec((BLKE, 64), row),
            pl.BlockSpec((PD, RELP), fixed),
            pl.BlockSpec((64, RELP), fixed),
            pl.BlockSpec((1, RELP), fixed),
        ],
        out_specs=pl.BlockSpec((BLKE, RELP), row),
        out_shape=jax.ShapeDtypeStruct((E, RELP), _F32),
    )(gs, s2, mw, spt, Wr1p, Wr2p, brel)


def kernel(obj_feats, obj_ctx, pos_embed, union_feats, spt_feats, pair_idx,
           obj_labels, W_low, b_low, W_high, b_high, W_s, b_s, W_o, b_o,
           W_ws, b_ws, W_wo, W_wu, W_hmp, b_hmp, W_rel, b_rel):
    ind = jnp.arange(1, NUM_OBJ + 1, dtype=_F32)[:, None]
    lin = jnp.linspace(-np.pi, np.pi, ORT_DIMS, dtype=_F32)[None, :]
    t = ind * lin
    ortp = jnp.zeros((256, ORT_DIMS), _F32).at[:NUM_OBJ].set(
        jnp.sin(t) + jnp.cos(t))

    pad_n = NP_ - N
    ofeat = jnp.pad(obj_feats, ((0, pad_n), (0, 0)))
    octx = jnp.pad(obj_ctx, ((0, pad_n), (0, 0)))
    posp = jnp.pad(pos_embed, ((0, pad_n), (0, 3)))
    lab = jnp.pad(obj_labels.astype(jnp.int32), (0, pad_n))[:, None]

    src = pair_idx[:, 0].astype(jnp.int32)
    dst = pair_idx[:, 1].astype(jnp.int32)

    Wr1p = jnp.pad(W_rel[:PD], ((0, 0), (0, RELP - 51)))
    Wr1w = jnp.pad(W_rel[:PD], ((0, 0), (0, RELW - 51)))
    Wr2p = jnp.pad(W_rel[PD:], ((0, 0), (0, RELP - 51)))
    brelp = jnp.pad(b_rel, (0, RELP - 51))[None, :]
    w_hmp = W_hmp.reshape(1, HD)
    b_hmp2 = b_hmp.reshape(1, 1)
    zer = jnp.zeros((NP_, RELW), _F32)

    sub, obj, hs, ho = _node_stage(
        ofeat, octx, posp, lab, ortp, W_low, b_low[None, :], W_high,
        b_high[None, :], W_s, b_s[None, :], W_o, b_o[None, :], W_ws,
        b_ws[None, :], W_wo)

    g1, g2 = _gather2(hs, ho, src, dst)

    mw = _edge1(union_feats, g1, g2, W_wu, w_hmp, b_hmp2, Wr1w)

    parts = _scatter(mw, src, zer)
    tsrc = _combine(parts, sub)

    gs, s2 = _gather3(tsrc, obj, src, dst)

    out64 = _edge2(gs, s2, mw, spt_feats, Wr1p, Wr2p, brelp)
    return out64[:, :51]


# speedup vs baseline: 1.7813x; 1.2314x over previous
import functools

import jax
import jax.numpy as jnp
import numpy as np
from jax import lax
from jax.experimental import pallas as pl
from jax.experimental.pallas import tpu as pltpu
from jax.experimental.pallas import tpu_sc as plsc

N = 10000
E = 160000
PD = 256
HD = 512
NUM_OBJ = 151
ORT_DIMS = 200
RELP = 64
RELW = 128
SRCW = PD // 2 + RELW
NP_ = 10240

NC, NS = 2, 16
NW = NC * NS
EPW = E // NW
CH = 40
NCHUNK = EPW // CH

BLKN = 1024
BLKE = 2000

_F32 = jnp.float32

_sc_mesh = plsc.VectorSubcoreMesh(
    core_axis_name="c", subcore_axis_name="s", num_cores=NC, num_subcores=NS)


def _wid():
    return lax.axis_index("s") * NC + lax.axis_index("c")


def _pack_bf16(x):
    w = x.shape[1] // 2
    xb = x.astype(jnp.bfloat16)
    lo = lax.bitcast_convert_type(xb[:, :w], jnp.uint16).astype(jnp.uint32)
    hi = lax.bitcast_convert_type(xb[:, w:], jnp.uint16).astype(jnp.uint32)
    return lax.bitcast_convert_type(lo | (hi << 16), jnp.int32)


def _unpack_bf16(p):
    u = lax.bitcast_convert_type(p, jnp.uint32)
    lo = lax.bitcast_convert_type((u & 0xFFFF).astype(jnp.uint16),
                                  jnp.bfloat16).astype(_F32)
    hi = lax.bitcast_convert_type((u >> 16).astype(jnp.uint16),
                                  jnp.bfloat16).astype(_F32)
    return jnp.concatenate([lo, hi], axis=1)


def _node_body(ofeat, octx, posp, lab, ortp,
               W_low, b_low, W_high, b_high, W_s, b_s, W_o, b_o, W_ws, b_ws,
               W_wo, sub_o, obj_o, hs_o, ho_o):
    wl = W_low[...]
    label_t = jnp.dot(ortp[...], wl[261:461, :],
                      preferred_element_type=_F32)
    iot = lax.broadcasted_iota(jnp.int32, (BLKN, 256), 1)
    oh = (iot == lab[...]).astype(_F32)
    rep = (jnp.dot(ofeat[...], wl[:256, :], preferred_element_type=_F32)
           + jnp.dot(posp[...], wl[256:264, :], preferred_element_type=_F32)
           + jnp.dot(oh, label_t, preferred_element_type=_F32)
           + b_low[...]
           + jnp.dot(octx[...], W_high[...], preferred_element_type=_F32)
           + b_high[...])
    sub = jnp.dot(rep, W_s[...], preferred_element_type=_F32) + b_s[...]
    obj = jnp.dot(rep, W_o[...], preferred_element_type=_F32) + b_o[...]
    sub_o[...] = _pack_bf16(sub)
    obj_o[...] = _pack_bf16(obj)
    hs_o[...] = _pack_bf16(
        jnp.dot(sub, W_ws[...], preferred_element_type=_F32))
    ho_o[...] = _pack_bf16(
        jnp.dot(obj, W_wo[...], preferred_element_type=_F32) + b_ws[...])


def _node_stage(ofeat, octx, posp, lab, ortp, W_low, b_low, W_high, b_high,
                W_s, b_s, W_o, b_o, W_ws, b_ws, W_wo):
    grid = NP_ // BLKN
    row = lambda i: (i, 0)
    fixed = lambda i: (0, 0)
    return pl.pallas_call(
        _node_body,
        grid=(grid,),
        in_specs=[
            pl.BlockSpec((BLKN, PD), row),
            pl.BlockSpec((BLKN, HD), row),
            pl.BlockSpec((BLKN, 8), row),
            pl.BlockSpec((BLKN, 1), row),
            pl.BlockSpec((256, ORT_DIMS), fixed),
            pl.BlockSpec((461, PD), fixed),
            pl.BlockSpec((1, PD), fixed),
            pl.BlockSpec((HD, PD), fixed),
            pl.BlockSpec((1, PD), fixed),
            pl.BlockSpec((PD, PD), fixed),
            pl.BlockSpec((1, PD), fixed),
            pl.BlockSpec((PD, PD), fixed),
            pl.BlockSpec((1, PD), fixed),
            pl.BlockSpec((PD, HD), fixed),
            pl.BlockSpec((1, HD), fixed),
            pl.BlockSpec((PD, HD), fixed),
        ],
        out_specs=[
            pl.BlockSpec((BLKN, PD // 2), row),
            pl.BlockSpec((BLKN, PD // 2), row),
            pl.BlockSpec((BLKN, HD // 2), row),
            pl.BlockSpec((BLKN, HD // 2), row),
        ],
        out_shape=[
            jax.ShapeDtypeStruct((NP_, PD // 2), jnp.int32),
            jax.ShapeDtypeStruct((NP_, PD // 2), jnp.int32),
            jax.ShapeDtypeStruct((NP_, HD // 2), jnp.int32),
            jax.ShapeDtypeStruct((NP_, HD // 2), jnp.int32),
        ],
    )(ofeat, octx, posp, lab, ortp, W_low, b_low, W_high, b_high,
      W_s, b_s, W_o, b_o, W_ws, b_ws, W_wo)


def _gather2_body(hs, ho, src, dst, g1, g2, idx_s, idx_d, buf1, buf2, sem):
    base = _wid() * EPW

    def step(i, _):
        off = base + i * CH
        pltpu.sync_copy(src.at[pl.ds(off, CH)], idx_s)
        pltpu.sync_copy(dst.at[pl.ds(off, CH)], idx_d)
        c1 = pltpu.async_copy(hs.at[idx_s], buf1, sem)
        c2 = pltpu.async_copy(ho.at[idx_d], buf2, sem)
        c1.wait()
        c2.wait()
        pltpu.sync_copy(buf1, g1.at[pl.ds(off, CH)])
        pltpu.sync_copy(buf2, g2.at[pl.ds(off, CH)])
        return 0

    lax.fori_loop(0, NCHUNK, step, 0)


_gather2 = functools.partial(
    pl.kernel,
    out_type=[jax.ShapeDtypeStruct((E, HD // 2), jnp.int32),
              jax.ShapeDtypeStruct((E, HD // 2), jnp.int32)],
    mesh=_sc_mesh,
    scratch_types=[
        pltpu.VMEM((CH,), jnp.int32),
        pltpu.VMEM((CH,), jnp.int32),
        pltpu.VMEM((CH, HD // 2), jnp.int32),
        pltpu.VMEM((CH, HD // 2), jnp.int32),
        pltpu.SemaphoreType.DMA,
    ],
)(_gather2_body)


def _edge1_body(u, g1, g2, W_wu, w_hmp, b_hmp, Wr1p, mw_o):
    ub = u[...].astype(jnp.bfloat16)
    wb = W_wu[...].astype(jnp.bfloat16)
    t = jnp.dot(ub, wb, preferred_element_type=_F32)
    h = jnp.tanh(_unpack_bf16(g1[...]) + _unpack_bf16(g2[...]) + t)
    logit = jnp.sum(h * w_hmp[...], axis=1, keepdims=True) + b_hmp[...]
    att = jax.nn.sigmoid(logit)
    r1b = Wr1p[...].astype(jnp.bfloat16)
    mw_o[...] = att * jnp.dot(ub, r1b, preferred_element_type=_F32)


def _edge1(u, g1, g2, W_wu, w_hmp, b_hmp, Wr1p):
    grid = E // BLKE
    row = lambda i: (i, 0)
    fixed = lambda i: (0, 0)
    return pl.pallas_call(
        _edge1_body,
        grid=(grid,),
        in_specs=[
            pl.BlockSpec((BLKE, PD), row),
            pl.BlockSpec((BLKE, HD // 2), row),
            pl.BlockSpec((BLKE, HD // 2), row),
            pl.BlockSpec((PD, HD), fixed),
            pl.BlockSpec((1, HD), fixed),
            pl.BlockSpec((1, 1), fixed),
            pl.BlockSpec((PD, RELW), fixed),
        ],
        out_specs=pl.BlockSpec((BLKE, RELW), row),
        out_shape=jax.ShapeDtypeStruct((E, RELW), _F32),
    )(u, g1, g2, W_wu, w_hmp, b_hmp, Wr1p)


def _scatter_body(mw, src, zer, out, idx_v, buf, aggsh):
    cid = lax.axis_index("c")
    sid = lax.axis_index("s")
    wid = sid * NC + cid
    rps = NP_ // NS
    pltpu.sync_copy(zer.at[pl.ds(sid * rps, rps)],
                    aggsh.at[pl.ds(sid * rps, rps)])
    plsc.subcore_barrier()
    base = wid * EPW

    def step(i, _):
        off = base + i * CH
        pltpu.sync_copy(src.at[pl.ds(off, CH)], idx_v)
        pltpu.sync_copy(mw.at[pl.ds(off, CH)], buf)
        pltpu.sync_copy(buf, aggsh.at[idx_v], add=True)
        return 0

    lax.fori_loop(0, NCHUNK, step, 0)
    plsc.subcore_barrier()
    pltpu.sync_copy(aggsh.at[pl.ds(sid * rps, rps)],
                    out.at[pl.ds(cid * NP_ + sid * rps, rps)])


_scatter = functools.partial(
    pl.kernel,
    out_type=jax.ShapeDtypeStruct((2 * NP_, RELW), _F32),
    mesh=_sc_mesh,
    scratch_types=[
        pltpu.VMEM((CH,), jnp.int32),
        pltpu.VMEM((CH, RELW), _F32),
        pltpu.VMEM_SHARED((NP_, RELW), _F32),
    ],
)(_scatter_body)


def _combine_body(p0, p1, sub, out_o):
    agg_i = lax.bitcast_convert_type(p0[...] + p1[...], jnp.int32)
    out_o[...] = jnp.concatenate([sub[...], agg_i], axis=1)


def _combine(parts, sub):
    grid = NP_ // BLKN
    row = lambda i: (i, 0)
    return pl.pallas_call(
        _combine_body,
        grid=(grid,),
        in_specs=[pl.BlockSpec((BLKN, RELW), row),
                  pl.BlockSpec((BLKN, RELW), row),
                  pl.BlockSpec((BLKN, PD // 2), row)],
        out_specs=pl.BlockSpec((BLKN, SRCW), row),
        out_shape=jax.ShapeDtypeStruct((NP_, SRCW), jnp.int32),
    )(parts[:NP_], parts[NP_:], sub)


def _gather3_body(tsrc, obj, src, dst, gs, s2, idx_s, idx_d, b1, b2, sem):
    base = _wid() * EPW

    def step(i, _):
        off = base + i * CH
        pltpu.sync_copy(src.at[pl.ds(off, CH)], idx_s)
        pltpu.sync_copy(dst.at[pl.ds(off, CH)], idx_d)
        c1 = pltpu.async_copy(tsrc.at[idx_s], b1, sem)
        c2 = pltpu.async_copy(obj.at[idx_d], b2, sem)
        c1.wait()
        c2.wait()
        pltpu.sync_copy(b1, gs.at[pl.ds(off, CH)])
        pltpu.sync_copy(b2, s2.at[pl.ds(off, CH)])
        return 0

    lax.fori_loop(0, NCHUNK, step, 0)


_gather3 = functools.partial(
    pl.kernel,
    out_type=[jax.ShapeDtypeStruct((E, SRCW), jnp.int32),
              jax.ShapeDtypeStruct((E, PD // 2), jnp.int32)],
    mesh=_sc_mesh,
    scratch_types=[
        pltpu.VMEM((CH,), jnp.int32),
        pltpu.VMEM((CH,), jnp.int32),
        pltpu.VMEM((CH, SRCW), jnp.int32),
        pltpu.VMEM((CH, PD // 2), jnp.int32),
        pltpu.SemaphoreType.DMA,
    ],
)(_gather3_body)


def _edge2_body(gs, s2, mw, spt, Wr1p, Wr2p, brel, out_o):
    g = gs[...]
    s1 = _unpack_bf16(g[:, :PD // 2])
    ag = lax.bitcast_convert_type(g[:, PD // 2:], _F32)
    p = s1 * _unpack_bf16(s2[...])
    out_o[...] = (jnp.dot(p, Wr1p[...], preferred_element_type=_F32)
                  + jnp.dot(spt[...], Wr2p[...], preferred_element_type=_F32)
                  + ag[:, :RELP] + mw[...][:, :RELP] + brel[...])


def _edge2(gs, s2, mw, spt, Wr1p, Wr2p, brel):
    grid = E // BLKE
    row = lambda i: (i, 0)
    fixed = lambda i: (0, 0)
    return pl.pallas_call(
        _edge2_body,
        grid=(grid,),
        in_specs=[
            pl.BlockSpec((BLKE, SRCW), row),
            pl.BlockSpec((BLKE, PD // 2), row),
            pl.BlockSpec((BLKE, RELW), row),
            pl.BlockSpec((BLKE, 64), row),
            pl.BlockSpec((PD, RELP), fixed),
            pl.BlockSpec((64, RELP), fixed),
            pl.BlockSpec((1, RELP), fixed),
        ],
        out_specs=pl.BlockSpec((BLKE, RELP), row),
        out_shape=jax.ShapeDtypeStruct((E, RELP), _F32),
    )(gs, s2, mw, spt, Wr1p, Wr2p, brel)


def kernel(obj_feats, obj_ctx, pos_embed, union_feats, spt_feats, pair_idx,
           obj_labels, W_low, b_low, W_high, b_high, W_s, b_s, W_o, b_o,
           W_ws, b_ws, W_wo, W_wu, W_hmp, b_hmp, W_rel, b_rel):
    ind = jnp.arange(1, NUM_OBJ + 1, dtype=_F32)[:, None]
    lin = jnp.linspace(-np.pi, np.pi, ORT_DIMS, dtype=_F32)[None, :]
    t = ind * lin
    ortp = jnp.zeros((256, ORT_DIMS), _F32).at[:NUM_OBJ].set(
        jnp.sin(t) + jnp.cos(t))

    pad_n = NP_ - N
    ofeat = jnp.pad(obj_feats, ((0, pad_n), (0, 0)))
    octx = jnp.pad(obj_ctx, ((0, pad_n), (0, 0)))
    posp = jnp.pad(pos_embed, ((0, pad_n), (0, 3)))
    lab = jnp.pad(obj_labels.astype(jnp.int32), (0, pad_n))[:, None]

    src = pair_idx[:, 0].astype(jnp.int32)
    dst = pair_idx[:, 1].astype(jnp.int32)

    Wr1p = jnp.pad(W_rel[:PD], ((0, 0), (0, RELP - 51)))
    Wr1w = jnp.pad(W_rel[:PD], ((0, 0), (0, RELW - 51)))
    Wr2p = jnp.pad(W_rel[PD:], ((0, 0), (0, RELP - 51)))
    brelp = jnp.pad(b_rel, (0, RELP - 51))[None, :]
    w_hmp = W_hmp.reshape(1, HD)
    b_hmp2 = b_hmp.reshape(1, 1)
    zer = jnp.zeros((NP_, RELW), _F32)

    sub, obj, hs, ho = _node_stage(
        ofeat, octx, posp, lab, ortp, W_low, b_low[None, :], W_high,
        b_high[None, :], W_s, b_s[None, :], W_o, b_o[None, :], W_ws,
        b_ws[None, :], W_wo)

    g1, g2 = _gather2(hs, ho, src, dst)

    mw = _edge1(union_feats, g1, g2, W_wu, w_hmp, b_hmp2, Wr1w)

    parts = _scatter(mw, src, zer)
    tsrc = _combine(parts, sub)

    gs, s2 = _gather3(tsrc, obj, src, dst)

    out64 = _edge2(gs, s2, mw, spt_feats, Wr1p, Wr2p, brelp)
    return out64[:, :51]


# speedup vs baseline: 2.4327x; 1.3656x over previous
import functools

import jax
import jax.numpy as jnp
import numpy as np
from jax import lax
from jax.experimental import pallas as pl
from jax.experimental.pallas import tpu as pltpu
from jax.experimental.pallas import tpu_sc as plsc

N = 10000
E = 160000
PD = 256
HD = 512
NUM_OBJ = 151
ORT_DIMS = 200
RELP = 64
RELW = 128
SRCW = PD // 2 + RELW
NP_ = 10240

NC, NS = 2, 16
NW = NC * NS
EPW = E // NW
CH = 200
NCHUNK = EPW // CH

BLKN = 1024
BLKE = 2000

_F32 = jnp.float32

_sc_mesh = plsc.VectorSubcoreMesh(
    core_axis_name="c", subcore_axis_name="s", num_cores=NC, num_subcores=NS)


def _wid():
    return lax.axis_index("s") * NC + lax.axis_index("c")


def _pack_bf16(x):
    w = x.shape[1] // 2
    xb = x.astype(jnp.bfloat16)
    lo = lax.bitcast_convert_type(xb[:, :w], jnp.uint16).astype(jnp.uint32)
    hi = lax.bitcast_convert_type(xb[:, w:], jnp.uint16).astype(jnp.uint32)
    return lax.bitcast_convert_type(lo | (hi << 16), jnp.int32)


def _unpack_bf16(p):
    u = lax.bitcast_convert_type(p, jnp.uint32)
    lo = lax.bitcast_convert_type((u & 0xFFFF).astype(jnp.uint16),
                                  jnp.bfloat16).astype(_F32)
    hi = lax.bitcast_convert_type((u >> 16).astype(jnp.uint16),
                                  jnp.bfloat16).astype(_F32)
    return jnp.concatenate([lo, hi], axis=1)


def _node_body(ofeat, octx, posp, lab, ortp,
               W_low, b_low, W_high, b_high, W_s, b_s, W_o, b_o, W_ws, b_ws,
               W_wo, sub_o, obj_o, hs_o, ho_o):
    wl = W_low[...]
    label_t = jnp.dot(ortp[...], wl[261:461, :],
                      preferred_element_type=_F32)
    iot = lax.broadcasted_iota(jnp.int32, (BLKN, 256), 1)
    oh = (iot == lab[...]).astype(_F32)
    rep = (jnp.dot(ofeat[...], wl[:256, :], preferred_element_type=_F32)
           + jnp.dot(posp[...], wl[256:264, :], preferred_element_type=_F32)
           + jnp.dot(oh, label_t, preferred_element_type=_F32)
           + b_low[...]
           + jnp.dot(octx[...], W_high[...], preferred_element_type=_F32)
           + b_high[...])
    sub = jnp.dot(rep, W_s[...], preferred_element_type=_F32) + b_s[...]
    obj = jnp.dot(rep, W_o[...], preferred_element_type=_F32) + b_o[...]
    sub_o[...] = _pack_bf16(sub)
    obj_o[...] = _pack_bf16(obj)
    hs_o[...] = _pack_bf16(
        jnp.dot(sub, W_ws[...], preferred_element_type=_F32))
    ho_o[...] = _pack_bf16(
        jnp.dot(obj, W_wo[...], preferred_element_type=_F32) + b_ws[...])


def _node_stage(ofeat, octx, posp, lab, ortp, W_low, b_low, W_high, b_high,
                W_s, b_s, W_o, b_o, W_ws, b_ws, W_wo):
    grid = NP_ // BLKN
    row = lambda i: (i, 0)
    fixed = lambda i: (0, 0)
    return pl.pallas_call(
        _node_body,
        grid=(grid,),
        in_specs=[
            pl.BlockSpec((BLKN, PD), row),
            pl.BlockSpec((BLKN, HD), row),
            pl.BlockSpec((BLKN, 8), row),
            pl.BlockSpec((BLKN, 1), row),
            pl.BlockSpec((256, ORT_DIMS), fixed),
            pl.BlockSpec((461, PD), fixed),
            pl.BlockSpec((1, PD), fixed),
            pl.BlockSpec((HD, PD), fixed),
            pl.BlockSpec((1, PD), fixed),
            pl.BlockSpec((PD, PD), fixed),
            pl.BlockSpec((1, PD), fixed),
            pl.BlockSpec((PD, PD), fixed),
            pl.BlockSpec((1, PD), fixed),
            pl.BlockSpec((PD, HD), fixed),
            pl.BlockSpec((1, HD), fixed),
            pl.BlockSpec((PD, HD), fixed),
        ],
        out_specs=[
            pl.BlockSpec((BLKN, PD // 2), row),
            pl.BlockSpec((BLKN, PD // 2), row),
            pl.BlockSpec((BLKN, HD // 2), row),
            pl.BlockSpec((BLKN, HD // 2), row),
        ],
        out_shape=[
            jax.ShapeDtypeStruct((NP_, PD // 2), jnp.int32),
            jax.ShapeDtypeStruct((NP_, PD // 2), jnp.int32),
            jax.ShapeDtypeStruct((NP_, HD // 2), jnp.int32),
            jax.ShapeDtypeStruct((NP_, HD // 2), jnp.int32),
        ],
    )(ofeat, octx, posp, lab, ortp, W_low, b_low, W_high, b_high,
      W_s, b_s, W_o, b_o, W_ws, b_ws, W_wo)


def _gather2_body(hs, ho, src, dst, g1, g2, idx_s, idx_d, buf1, buf2, sem):
    base = _wid() * EPW

    def step(i, _):
        off = base + i * CH
        pltpu.sync_copy(src.at[pl.ds(off, CH)], idx_s)
        pltpu.sync_copy(dst.at[pl.ds(off, CH)], idx_d)
        c1 = pltpu.async_copy(hs.at[idx_s], buf1, sem)
        c2 = pltpu.async_copy(ho.at[idx_d], buf2, sem)
        c1.wait()
        c2.wait()
        pltpu.sync_copy(buf1, g1.at[pl.ds(off, CH)])
        pltpu.sync_copy(buf2, g2.at[pl.ds(off, CH)])
        return 0

    lax.fori_loop(0, NCHUNK, step, 0)


_gather2 = functools.partial(
    pl.kernel,
    out_type=[jax.ShapeDtypeStruct((E, HD // 2), jnp.int32),
              jax.ShapeDtypeStruct((E, HD // 2), jnp.int32)],
    mesh=_sc_mesh,
    scratch_types=[
        pltpu.VMEM((CH,), jnp.int32),
        pltpu.VMEM((CH,), jnp.int32),
        pltpu.VMEM((CH, HD // 2), jnp.int32),
        pltpu.VMEM((CH, HD // 2), jnp.int32),
        pltpu.SemaphoreType.DMA,
    ],
)(_gather2_body)


def _edge1_body(u, g1, g2, W_wu, w_hmp, b_hmp, Wr1p, mw_o):
    ub = u[...].astype(jnp.bfloat16)
    wb = W_wu[...].astype(jnp.bfloat16)
    t = jnp.dot(ub, wb, preferred_element_type=_F32)
    h = jnp.tanh(_unpack_bf16(g1[...]) + _unpack_bf16(g2[...]) + t)
    logit = jnp.sum(h * w_hmp[...], axis=1, keepdims=True) + b_hmp[...]
    att = jax.nn.sigmoid(logit)
    r1b = Wr1p[...].astype(jnp.bfloat16)
    mw_o[...] = att * jnp.dot(ub, r1b, preferred_element_type=_F32)


def _edge1(u, g1, g2, W_wu, w_hmp, b_hmp, Wr1p):
    grid = E // BLKE
    row = lambda i: (i, 0)
    fixed = lambda i: (0, 0)
    return pl.pallas_call(
        _edge1_body,
        grid=(grid,),
        in_specs=[
            pl.BlockSpec((BLKE, PD), row),
            pl.BlockSpec((BLKE, HD // 2), row),
            pl.BlockSpec((BLKE, HD // 2), row),
            pl.BlockSpec((PD, HD), fixed),
            pl.BlockSpec((1, HD), fixed),
            pl.BlockSpec((1, 1), fixed),
            pl.BlockSpec((PD, RELW), fixed),
        ],
        out_specs=pl.BlockSpec((BLKE, RELW), row),
        out_shape=jax.ShapeDtypeStruct((E, RELW), _F32),
    )(u, g1, g2, W_wu, w_hmp, b_hmp, Wr1p)


def _scatter_body(mw, src, zer, out, idx_v, buf, aggsh):
    cid = lax.axis_index("c")
    sid = lax.axis_index("s")
    wid = sid * NC + cid
    rps = NP_ // NS
    pltpu.sync_copy(zer.at[pl.ds(sid * rps, rps)],
                    aggsh.at[pl.ds(sid * rps, rps)])
    plsc.subcore_barrier()
    base = wid * EPW

    def step(i, _):
        off = base + i * CH
        pltpu.sync_copy(src.at[pl.ds(off, CH)], idx_v)
        pltpu.sync_copy(mw.at[pl.ds(off, CH)], buf)
        pltpu.sync_copy(buf, aggsh.at[idx_v], add=True)
        return 0

    lax.fori_loop(0, NCHUNK, step, 0)
    plsc.subcore_barrier()
    pltpu.sync_copy(aggsh.at[pl.ds(sid * rps, rps)],
                    out.at[pl.ds(cid * NP_ + sid * rps, rps)])


_scatter = functools.partial(
    pl.kernel,
    out_type=jax.ShapeDtypeStruct((2 * NP_, RELW), _F32),
    mesh=_sc_mesh,
    scratch_types=[
        pltpu.VMEM((CH,), jnp.int32),
        pltpu.VMEM((CH, RELW), _F32),
        pltpu.VMEM_SHARED((NP_, RELW), _F32),
    ],
)(_scatter_body)


def _combine_body(p0, p1, sub, out_o):
    agg_i = lax.bitcast_convert_type(p0[...] + p1[...], jnp.int32)
    out_o[...] = jnp.concatenate([sub[...], agg_i], axis=1)


def _combine(parts, sub):
    grid = NP_ // BLKN
    row = lambda i: (i, 0)
    return pl.pallas_call(
        _combine_body,
        grid=(grid,),
        in_specs=[pl.BlockSpec((BLKN, RELW), row),
                  pl.BlockSpec((BLKN, RELW), row),
                  pl.BlockSpec((BLKN, PD // 2), row)],
        out_specs=pl.BlockSpec((BLKN, SRCW), row),
        out_shape=jax.ShapeDtypeStruct((NP_, SRCW), jnp.int32),
    )(parts[:NP_], parts[NP_:], sub)


def _gather3_body(tsrc, obj, src, dst, gs, s2, idx_s, idx_d, b1, b2, sem):
    base = _wid() * EPW

    def step(i, _):
        off = base + i * CH
        pltpu.sync_copy(src.at[pl.ds(off, CH)], idx_s)
        pltpu.sync_copy(dst.at[pl.ds(off, CH)], idx_d)
        c1 = pltpu.async_copy(tsrc.at[idx_s], b1, sem)
        c2 = pltpu.async_copy(obj.at[idx_d], b2, sem)
        c1.wait()
        c2.wait()
        pltpu.sync_copy(b1, gs.at[pl.ds(off, CH)])
        pltpu.sync_copy(b2, s2.at[pl.ds(off, CH)])
        return 0

    lax.fori_loop(0, NCHUNK, step, 0)


_gather3 = functools.partial(
    pl.kernel,
    out_type=[jax.ShapeDtypeStruct((E, SRCW), jnp.int32),
              jax.ShapeDtypeStruct((E, PD // 2), jnp.int32)],
    mesh=_sc_mesh,
    scratch_types=[
        pltpu.VMEM((CH,), jnp.int32),
        pltpu.VMEM((CH,), jnp.int32),
        pltpu.VMEM((CH, SRCW), jnp.int32),
        pltpu.VMEM((CH, PD // 2), jnp.int32),
        pltpu.SemaphoreType.DMA,
    ],
)(_gather3_body)


def _edge2_body(gs, s2, mw, spt, Wr1p, Wr2p, brel, out_o):
    g = gs[...]
    s1 = _unpack_bf16(g[:, :PD // 2])
    ag = lax.bitcast_convert_type(g[:, PD // 2:], _F32)
    p = s1 * _unpack_bf16(s2[...])
    out_o[...] = (jnp.dot(p, Wr1p[...], preferred_element_type=_F32)
                  + jnp.dot(spt[...], Wr2p[...], preferred_element_type=_F32)
                  + ag[:, :RELP] + mw[...][:, :RELP] + brel[...])


def _edge2(gs, s2, mw, spt, Wr1p, Wr2p, brel):
    grid = E // BLKE
    row = lambda i: (i, 0)
    fixed = lambda i: (0, 0)
    return pl.pallas_call(
        _edge2_body,
        grid=(grid,),
        in_specs=[
            pl.BlockSpec((BLKE, SRCW), row),
            pl.BlockSpec((BLKE, PD // 2), row),
            pl.BlockSpec((BLKE, RELW), row),
            pl.BlockSpec((BLKE, 64), row),
            pl.BlockSpec((PD, RELP), fixed),
            pl.BlockSpec((64, RELP), fixed),
            pl.BlockSpec((1, RELP), fixed),
        ],
        out_specs=pl.BlockSpec((BLKE, RELP), row),
        out_shape=jax.ShapeDtypeStruct((E, RELP), _F32),
    )(gs, s2, mw, spt, Wr1p, Wr2p, brel)


def kernel(obj_feats, obj_ctx, pos_embed, union_feats, spt_feats, pair_idx,
           obj_labels, W_low, b_low, W_high, b_high, W_s, b_s, W_o, b_o,
           W_ws, b_ws, W_wo, W_wu, W_hmp, b_hmp, W_rel, b_rel):
    ind = jnp.arange(1, NUM_OBJ + 1, dtype=_F32)[:, None]
    lin = jnp.linspace(-np.pi, np.pi, ORT_DIMS, dtype=_F32)[None, :]
    t = ind * lin
    ortp = jnp.zeros((256, ORT_DIMS), _F32).at[:NUM_OBJ].set(
        jnp.sin(t) + jnp.cos(t))

    pad_n = NP_ - N
    ofeat = jnp.pad(obj_feats, ((0, pad_n), (0, 0)))
    octx = jnp.pad(obj_ctx, ((0, pad_n), (0, 0)))
    posp = jnp.pad(pos_embed, ((0, pad_n), (0, 3)))
    lab = jnp.pad(obj_labels.astype(jnp.int32), (0, pad_n))[:, None]

    src = pair_idx[:, 0].astype(jnp.int32)
    dst = pair_idx[:, 1].astype(jnp.int32)

    Wr1p = jnp.pad(W_rel[:PD], ((0, 0), (0, RELP - 51)))
    Wr1w = jnp.pad(W_rel[:PD], ((0, 0), (0, RELW - 51)))
    Wr2p = jnp.pad(W_rel[PD:], ((0, 0), (0, RELP - 51)))
    brelp = jnp.pad(b_rel, (0, RELP - 51))[None, :]
    w_hmp = W_hmp.reshape(1, HD)
    b_hmp2 = b_hmp.reshape(1, 1)
    zer = jnp.zeros((NP_, RELW), _F32)

    sub, obj, hs, ho = _node_stage(
        ofeat, octx, posp, lab, ortp, W_low, b_low[None, :], W_high,
        b_high[None, :], W_s, b_s[None, :], W_o, b_o[None, :], W_ws,
        b_ws[None, :], W_wo)

    g1, g2 = _gather2(hs, ho, src, dst)

    mw = _edge1(union_feats, g1, g2, W_wu, w_hmp, b_hmp2, Wr1w)

    parts = _scatter(mw, src, zer)
    tsrc = _combine(parts, sub)

    gs, s2 = _gather3(tsrc, obj, src, dst)

    out64 = _edge2(gs, s2, mw, spt_feats, Wr1p, Wr2p, brelp)
    return out64[:, :51]


# speedup vs baseline: 2.6243x; 1.0788x over previous
import functools

import jax
import jax.numpy as jnp
import numpy as np
from jax import lax
from jax.experimental import pallas as pl
from jax.experimental.pallas import tpu as pltpu
from jax.experimental.pallas import tpu_sc as plsc

N = 10000
E = 160000
PD = 256
HD = 512
NUM_OBJ = 151
ORT_DIMS = 200
NREL = 51
RELP = 64
RELW = 128
SRCW = PD // 2 + RELW
NP_ = 10240

NC, NS = 2, 16
NW = NC * NS
CH = 200

BLKN = 1024
BLKE = 1600

SL_START = (0, 76800)
SL_E = (76800, 83200)
SL_EPW = (2400, 2600)
SL_NCH = (12, 13)
SL_BLK0 = (0, 48)

_F32 = jnp.float32

_sc_mesh = plsc.VectorSubcoreMesh(
    core_axis_name="c", subcore_axis_name="s", num_cores=NC, num_subcores=NS)


def _wid():
    return lax.axis_index("s") * NC + lax.axis_index("c")


def _pack_bf16(x):
    w = x.shape[1] // 2
    xb = x.astype(jnp.bfloat16)
    lo = lax.bitcast_convert_type(xb[:, :w], jnp.uint16).astype(jnp.uint32)
    hi = lax.bitcast_convert_type(xb[:, w:], jnp.uint16).astype(jnp.uint32)
    return lax.bitcast_convert_type(lo | (hi << 16), jnp.int32)


def _unpack_bf16(p):
    u = lax.bitcast_convert_type(p, jnp.uint32)
    lo = lax.bitcast_convert_type((u & 0xFFFF).astype(jnp.uint16),
                                  jnp.bfloat16).astype(_F32)
    hi = lax.bitcast_convert_type((u >> 16).astype(jnp.uint16),
                                  jnp.bfloat16).astype(_F32)
    return jnp.concatenate([lo, hi], axis=1)


def _node_body(ofeat, octx, posp, lab, ortp,
               W_low, b_low, W_high, b_high, W_s, b_s, W_o, b_o, W_ws, b_ws,
               W_wo, sub_o, obj_o, hs_o, ho_o):
    wl = W_low[...]
    label_t = jnp.dot(ortp[...], wl[261:461, :],
                      preferred_element_type=_F32)
    iot = lax.broadcasted_iota(jnp.int32, (BLKN, 256), 1)
    oh = (iot == lab[...]).astype(_F32)
    rep = (jnp.dot(ofeat[...], wl[:256, :], preferred_element_type=_F32)
           + jnp.dot(posp[...], wl[256:264, :], preferred_element_type=_F32)
           + jnp.dot(oh, label_t, preferred_element_type=_F32)
           + b_low[...]
           + jnp.dot(octx[...], W_high[...], preferred_element_type=_F32)
           + b_high[...])
    sub = jnp.dot(rep, W_s[...], preferred_element_type=_F32) + b_s[...]
    obj = jnp.dot(rep, W_o[...], preferred_element_type=_F32) + b_o[...]
    sub_o[...] = _pack_bf16(sub)
    obj_o[...] = _pack_bf16(obj)
    hs_o[...] = _pack_bf16(
        jnp.dot(sub, W_ws[...], preferred_element_type=_F32))
    ho_o[...] = _pack_bf16(
        jnp.dot(obj, W_wo[...], preferred_element_type=_F32) + b_ws[...])


def _node_stage(ofeat, octx, posp, lab, ortp, W_low, b_low, W_high, b_high,
                W_s, b_s, W_o, b_o, W_ws, b_ws, W_wo):
    row = lambda i: (i, 0)
    fixed = lambda i: (0, 0)
    return pl.pallas_call(
        _node_body,
        grid=(NP_ // BLKN,),
        in_specs=[
            pl.BlockSpec((BLKN, PD), row),
            pl.BlockSpec((BLKN, HD), row),
            pl.BlockSpec((BLKN, 8), row),
            pl.BlockSpec((BLKN, 1), row),
            pl.BlockSpec((256, ORT_DIMS), fixed),
            pl.BlockSpec((461, PD), fixed),
            pl.BlockSpec((1, PD), fixed),
            pl.BlockSpec((HD, PD), fixed),
            pl.BlockSpec((1, PD), fixed),
            pl.BlockSpec((PD, PD), fixed),
            pl.BlockSpec((1, PD), fixed),
            pl.BlockSpec((PD, PD), fixed),
            pl.BlockSpec((1, PD), fixed),
            pl.BlockSpec((PD, HD), fixed),
            pl.BlockSpec((1, HD), fixed),
            pl.BlockSpec((PD, HD), fixed),
        ],
        out_specs=[
            pl.BlockSpec((BLKN, PD // 2), row),
            pl.BlockSpec((BLKN, PD // 2), row),
            pl.BlockSpec((BLKN, HD // 2), row),
            pl.BlockSpec((BLKN, HD // 2), row),
        ],
        out_shape=[
            jax.ShapeDtypeStruct((NP_, PD // 2), jnp.int32),
            jax.ShapeDtypeStruct((NP_, PD // 2), jnp.int32),
            jax.ShapeDtypeStruct((NP_, HD // 2), jnp.int32),
            jax.ShapeDtypeStruct((NP_, HD // 2), jnp.int32),
        ],
    )(ofeat, octx, posp, lab, ortp, W_low, b_low, W_high, b_high,
      W_s, b_s, W_o, b_o, W_ws, b_ws, W_wo)


def _make_gather2(slab):
    start, epw, nch, esz = (SL_START[slab], SL_EPW[slab], SL_NCH[slab],
                            SL_E[slab])

    def body(hs, ho, src, dst, g1, g2, idx_s, idx_d, buf1, buf2, sem):
        base = _wid() * epw

        def step(i, _):
            off = base + i * CH
            pltpu.sync_copy(src.at[pl.ds(start + off, CH)], idx_s)
            pltpu.sync_copy(dst.at[pl.ds(start + off, CH)], idx_d)
            c1 = pltpu.async_copy(hs.at[idx_s], buf1, sem)
            c2 = pltpu.async_copy(ho.at[idx_d], buf2, sem)
            c1.wait()
            c2.wait()
            pltpu.sync_copy(buf1, g1.at[pl.ds(off, CH)])
            pltpu.sync_copy(buf2, g2.at[pl.ds(off, CH)])
            return 0

        lax.fori_loop(0, nch, step, 0)

    return functools.partial(
        pl.kernel,
        out_type=[jax.ShapeDtypeStruct((esz, HD // 2), jnp.int32),
                  jax.ShapeDtypeStruct((esz, HD // 2), jnp.int32)],
        mesh=_sc_mesh,
        scratch_types=[
            pltpu.VMEM((CH,), jnp.int32),
            pltpu.VMEM((CH,), jnp.int32),
            pltpu.VMEM((CH, HD // 2), jnp.int32),
            pltpu.VMEM((CH, HD // 2), jnp.int32),
            pltpu.SemaphoreType.DMA,
        ],
    )(body)


_gather2_s = [_make_gather2(0), _make_gather2(1)]


def _edge1_body(u, g1, g2, W_wu, w_hmp, b_hmp, Wr1p, mw_o):
    ub = u[...].astype(jnp.bfloat16)
    wb = W_wu[...].astype(jnp.bfloat16)
    t = jnp.dot(ub, wb, preferred_element_type=_F32)
    h = jnp.tanh(_unpack_bf16(g1[...]) + _unpack_bf16(g2[...]) + t)
    logit = jnp.sum(h * w_hmp[...], axis=1, keepdims=True) + b_hmp[...]
    att = jax.nn.sigmoid(logit)
    r1b = Wr1p[...].astype(jnp.bfloat16)
    mw_o[...] = att * jnp.dot(ub, r1b, preferred_element_type=_F32)


def _edge1(slab, u, g1, g2, W_wu, w_hmp, b_hmp, Wr1p):
    blk0, esz = SL_BLK0[slab], SL_E[slab]
    rowg = lambda i: (i + blk0, 0)
    row = lambda i: (i, 0)
    fixed = lambda i: (0, 0)
    return pl.pallas_call(
        _edge1_body,
        grid=(esz // BLKE,),
        in_specs=[
            pl.BlockSpec((BLKE, PD), rowg),
            pl.BlockSpec((BLKE, HD // 2), row),
            pl.BlockSpec((BLKE, HD // 2), row),
            pl.BlockSpec((PD, HD), fixed),
            pl.BlockSpec((1, HD), fixed),
            pl.BlockSpec((1, 1), fixed),
            pl.BlockSpec((PD, RELW), fixed),
        ],
        out_specs=pl.BlockSpec((BLKE, RELW), row),
        out_shape=jax.ShapeDtypeStruct((esz, RELW), _F32),
    )(u, g1, g2, W_wu, w_hmp, b_hmp, Wr1p)


def _make_scatter(slab):
    start, epw, nch = SL_START[slab], SL_EPW[slab], SL_NCH[slab]
    rps = NP_ // NS

    def body(mw, src, zer, out, idx_v, buf, aggsh):
        cid = lax.axis_index("c")
        sid = lax.axis_index("s")
        wid = sid * NC + cid
        pltpu.sync_copy(zer.at[pl.ds(sid * rps, rps)],
                        aggsh.at[pl.ds(sid * rps, rps)])
        plsc.subcore_barrier()
        base = wid * epw

        def step(i, _):
            off = base + i * CH
            pltpu.sync_copy(src.at[pl.ds(start + off, CH)], idx_v)
            pltpu.sync_copy(mw.at[pl.ds(off, CH)], buf)
            pltpu.sync_copy(buf, aggsh.at[idx_v], add=True)
            return 0

        lax.fori_loop(0, nch, step, 0)
        plsc.subcore_barrier()
        pltpu.sync_copy(aggsh.at[pl.ds(sid * rps, rps)],
                        out.at[pl.ds(cid * NP_ + sid * rps, rps)])

    return functools.partial(
        pl.kernel,
        out_type=jax.ShapeDtypeStruct((2 * NP_, RELW), _F32),
        mesh=_sc_mesh,
        scratch_types=[
            pltpu.VMEM((CH,), jnp.int32),
            pltpu.VMEM((CH, RELW), _F32),
            pltpu.VMEM_SHARED((NP_, RELW), _F32),
        ],
    )(body)


_scatter_s = [_make_scatter(0), _make_scatter(1)]


def _combine_body(pa0, pa1, pb0, pb1, sub, out_o):
    agg = pa0[...] + pa1[...] + pb0[...] + pb1[...]
    out_o[...] = jnp.concatenate(
        [sub[...], lax.bitcast_convert_type(agg, jnp.int32)], axis=1)


def _combine(parts_a, parts_b, sub):
    h0 = lambda i: (i, 0)
    h1 = lambda i: (i + NP_ // BLKN, 0)
    return pl.pallas_call(
        _combine_body,
        grid=(NP_ // BLKN,),
        in_specs=[pl.BlockSpec((BLKN, RELW), h0),
                  pl.BlockSpec((BLKN, RELW), h1),
                  pl.BlockSpec((BLKN, RELW), h0),
                  pl.BlockSpec((BLKN, RELW), h1),
                  pl.BlockSpec((BLKN, PD // 2), h0)],
        out_specs=pl.BlockSpec((BLKN, SRCW), h0),
        out_shape=jax.ShapeDtypeStruct((NP_, SRCW), jnp.int32),
    )(parts_a, parts_a, parts_b, parts_b, sub)


def _make_gather3(slab):
    start, epw, nch, esz = (SL_START[slab], SL_EPW[slab], SL_NCH[slab],
                            SL_E[slab])

    def body(tsrc, obj, src, dst, gs, s2, idx_s, idx_d, b1, b2, sem):
        base = _wid() * epw

        def step(i, _):
            off = base + i * CH
            pltpu.sync_copy(src.at[pl.ds(start + off, CH)], idx_s)
            pltpu.sync_copy(dst.at[pl.ds(start + off, CH)], idx_d)
            c1 = pltpu.async_copy(tsrc.at[idx_s], b1, sem)
            c2 = pltpu.async_copy(obj.at[idx_d], b2, sem)
            c1.wait()
            c2.wait()
            pltpu.sync_copy(b1, gs.at[pl.ds(off, CH)])
            pltpu.sync_copy(b2, s2.at[pl.ds(off, CH)])
            return 0

        lax.fori_loop(0, nch, step, 0)

    return functools.partial(
        pl.kernel,
        out_type=[jax.ShapeDtypeStruct((esz, SRCW), jnp.int32),
                  jax.ShapeDtypeStruct((esz, PD // 2), jnp.int32)],
        mesh=_sc_mesh,
        scratch_types=[
            pltpu.VMEM((CH,), jnp.int32),
            pltpu.VMEM((CH,), jnp.int32),
            pltpu.VMEM((CH, SRCW), jnp.int32),
            pltpu.VMEM((CH, PD // 2), jnp.int32),
            pltpu.SemaphoreType.DMA,
        ],
    )(body)


_gather3_s = [_make_gather3(0), _make_gather3(1)]


def _edge2_body(gs, s2, mw, spt, Wr1p, Wr2p, brel, out_o):
    g = gs[...]
    s1 = _unpack_bf16(g[:, :PD // 2])
    ag = lax.bitcast_convert_type(g[:, PD // 2:], _F32)
    p = s1 * _unpack_bf16(s2[...])
    full = (jnp.dot(p, Wr1p[...], preferred_element_type=_F32)
            + jnp.dot(spt[...], Wr2p[...], preferred_element_type=_F32)
            + ag[:, :RELP] + mw[...][:, :RELP] + brel[...])
    out_o[...] = full[:, :NREL]


def _edge2(slab, gs, s2, mw, spt, Wr1p, Wr2p, brel):
    blk0, esz = SL_BLK0[slab], SL_E[slab]
    rowg = lambda i: (i + blk0, 0)
    row = lambda i: (i, 0)
    fixed = lambda i: (0, 0)
    return pl.pallas_call(
        _edge2_body,
        grid=(esz // BLKE,),
        in_specs=[
            pl.BlockSpec((BLKE, SRCW), row),
            pl.BlockSpec((BLKE, PD // 2), row),
            pl.BlockSpec((BLKE, RELW), row),
            pl.BlockSpec((BLKE, 64), rowg),
            pl.BlockSpec((PD, RELP), fixed),
            pl.BlockSpec((64, RELP), fixed),
            pl.BlockSpec((1, RELP), fixed),
        ],
        out_specs=pl.BlockSpec((BLKE, NREL), row),
        out_shape=jax.ShapeDtypeStruct((esz, NREL), _F32),
    )(gs, s2, mw, spt, Wr1p, Wr2p, brel)


def kernel(obj_feats, obj_ctx, pos_embed, union_feats, spt_feats, pair_idx,
           obj_labels, W_low, b_low, W_high, b_high, W_s, b_s, W_o, b_o,
           W_ws, b_ws, W_wo, W_wu, W_hmp, b_hmp, W_rel, b_rel):
    ind = jnp.arange(1, NUM_OBJ + 1, dtype=_F32)[:, None]
    lin = jnp.linspace(-np.pi, np.pi, ORT_DIMS, dtype=_F32)[None, :]
    t = ind * lin
    ortp = jnp.zeros((256, ORT_DIMS), _F32).at[:NUM_OBJ].set(
        jnp.sin(t) + jnp.cos(t))

    pad_n = NP_ - N
    ofeat = jnp.pad(obj_feats, ((0, pad_n), (0, 0)))
    octx = jnp.pad(obj_ctx, ((0, pad_n), (0, 0)))
    posp = jnp.pad(pos_embed, ((0, pad_n), (0, 3)))
    lab = jnp.pad(obj_labels.astype(jnp.int32), (0, pad_n))[:, None]

    src = pair_idx[:, 0].astype(jnp.int32)
    dst = pair_idx[:, 1].astype(jnp.int32)

    Wr1p = jnp.pad(W_rel[:PD], ((0, 0), (0, RELP - NREL)))
    Wr1w = jnp.pad(W_rel[:PD], ((0, 0), (0, RELW - NREL)))
    Wr2p = jnp.pad(W_rel[PD:], ((0, 0), (0, RELP - NREL)))
    brelp = jnp.pad(b_rel, (0, RELP - NREL))[None, :]
    w_hmp = W_hmp.reshape(1, HD)
    b_hmp2 = b_hmp.reshape(1, 1)
    zer = jnp.zeros((NP_, RELW), _F32)

    sub_p, obj_p, hs_p, ho_p = _node_stage(
        ofeat, octx, posp, lab, ortp, W_low, b_low[None, :], W_high,
        b_high[None, :], W_s, b_s[None, :], W_o, b_o[None, :], W_ws,
        b_ws[None, :], W_wo)

    g1a, g2a = _gather2_s[0](hs_p, ho_p, src, dst)
    g1b, g2b = _gather2_s[1](hs_p, ho_p, src, dst)

    mwa = _edge1(0, union_feats, g1a, g2a, W_wu, w_hmp, b_hmp2, Wr1w)
    mwb = _edge1(1, union_feats, g1b, g2b, W_wu, w_hmp, b_hmp2, Wr1w)

    parts_a = _scatter_s[0](mwa, src, zer)
    parts_b = _scatter_s[1](mwb, src, zer)
    tsrc = _combine(parts_a, parts_b, sub_p)

    gsa, s2a = _gather3_s[0](tsrc, obj_p, src, dst)
    gsb, s2b = _gather3_s[1](tsrc, obj_p, src, dst)

    outa = _edge2(0, gsa, s2a, mwa, spt_feats, Wr1p, Wr2p, brelp)
    outb = _edge2(1, gsb, s2b, mwb, spt_feats, Wr1p, Wr2p, brelp)
    return jnp.concatenate([outa, outb], axis=0)


# speedup vs baseline: 2.9463x; 1.1227x over previous
import functools

import jax
import jax.numpy as jnp
import numpy as np
from jax import lax
from jax.experimental import pallas as pl
from jax.experimental.pallas import tpu as pltpu
from jax.experimental.pallas import tpu_sc as plsc

N = 10000
E = 160000
PD = 256
HD = 512
NUM_OBJ = 151
ORT_DIMS = 200
NREL = 51
RELP = 64
RELW = 128
SRCW = PD // 4 + RELP
HS_SCALE = 16.0
SUB_SCALE = 4.0
NP_ = 10240

NC, NS = 2, 16
NW = NC * NS
CH = 200

BLKN = 1024
BLKE = 1600

SL_START = (0, 76800)
SL_E = (76800, 83200)
SL_EPW = (2400, 2600)
SL_NCH = (12, 13)
SL_BLK0 = (0, 48)

_F32 = jnp.float32

_sc_mesh = plsc.VectorSubcoreMesh(
    core_axis_name="c", subcore_axis_name="s", num_cores=NC, num_subcores=NS)


def _wid():
    return lax.axis_index("s") * NC + lax.axis_index("c")


def _pack_bf16(x):
    w = x.shape[1] // 2
    xb = x.astype(jnp.bfloat16)
    lo = lax.bitcast_convert_type(xb[:, :w], jnp.uint16).astype(jnp.uint32)
    hi = lax.bitcast_convert_type(xb[:, w:], jnp.uint16).astype(jnp.uint32)
    return lax.bitcast_convert_type(lo | (hi << 16), jnp.int32)


def _unpack_bf16(p):
    u = lax.bitcast_convert_type(p, jnp.uint32)
    lo = lax.bitcast_convert_type((u & 0xFFFF).astype(jnp.uint16),
                                  jnp.bfloat16).astype(_F32)
    hi = lax.bitcast_convert_type((u >> 16).astype(jnp.uint16),
                                  jnp.bfloat16).astype(_F32)
    return jnp.concatenate([lo, hi], axis=1)


_F8 = jnp.float8_e4m3fn


def _pack_f8(x, scale):
    w = x.shape[1] // 4
    xb = (x * scale).astype(_F8)
    acc = None
    for k in range(4):
        b = lax.bitcast_convert_type(xb[:, k * w:(k + 1) * w],
                                     jnp.uint8).astype(jnp.uint32) << (8 * k)
        acc = b if acc is None else acc | b
    return lax.bitcast_convert_type(acc, jnp.int32)


def _unpack_f8(p, inv_scale):
    u = lax.bitcast_convert_type(p, jnp.uint32)
    parts = []
    for k in range(4):
        b = ((u >> (8 * k)) & 0xFF).astype(jnp.uint8)
        parts.append(
            lax.bitcast_convert_type(b, _F8).astype(_F32) * inv_scale)
    return jnp.concatenate(parts, axis=1)


def _node_body(ofeat, octx, posp, lab, ortp,
               W_low, b_low, W_high, b_high, W_s, b_s, W_o, b_o, W_ws, b_ws,
               W_wo, sub_o, obj_o, hs_o, ho_o):
    wl = W_low[...]
    label_t = jnp.dot(ortp[...], wl[261:461, :],
                      preferred_element_type=_F32)
    iot = lax.broadcasted_iota(jnp.int32, (BLKN, 256), 1)
    oh = (iot == lab[...]).astype(_F32)
    rep = (jnp.dot(ofeat[...], wl[:256, :], preferred_element_type=_F32)
           + jnp.dot(posp[...], wl[256:264, :], preferred_element_type=_F32)
           + jnp.dot(oh, label_t, preferred_element_type=_F32)
           + b_low[...]
           + jnp.dot(octx[...], W_high[...], preferred_element_type=_F32)
           + b_high[...])
    sub = jnp.dot(rep, W_s[...], preferred_element_type=_F32) + b_s[...]
    obj = jnp.dot(rep, W_o[...], preferred_element_type=_F32) + b_o[...]
    sub_o[...] = _pack_f8(sub, SUB_SCALE)
    obj_o[...] = _pack_bf16(obj)
    hs_o[...] = _pack_f8(
        jnp.dot(sub, W_ws[...], preferred_element_type=_F32), HS_SCALE)
    ho_o[...] = _pack_f8(
        jnp.dot(obj, W_wo[...], preferred_element_type=_F32) + b_ws[...],
        HS_SCALE)


def _node_stage(ofeat, octx, posp, lab, ortp, W_low, b_low, W_high, b_high,
                W_s, b_s, W_o, b_o, W_ws, b_ws, W_wo):
    row = lambda i: (i, 0)
    fixed = lambda i: (0, 0)
    return pl.pallas_call(
        _node_body,
        grid=(NP_ // BLKN,),
        in_specs=[
            pl.BlockSpec((BLKN, PD), row),
            pl.BlockSpec((BLKN, HD), row),
            pl.BlockSpec((BLKN, 8), row),
            pl.BlockSpec((BLKN, 1), row),
            pl.BlockSpec((256, ORT_DIMS), fixed),
            pl.BlockSpec((461, PD), fixed),
            pl.BlockSpec((1, PD), fixed),
            pl.BlockSpec((HD, PD), fixed),
            pl.BlockSpec((1, PD), fixed),
            pl.BlockSpec((PD, PD), fixed),
            pl.BlockSpec((1, PD), fixed),
            pl.BlockSpec((PD, PD), fixed),
            pl.BlockSpec((1, PD), fixed),
            pl.BlockSpec((PD, HD), fixed),
            pl.BlockSpec((1, HD), fixed),
            pl.BlockSpec((PD, HD), fixed),
        ],
        out_specs=[
            pl.BlockSpec((BLKN, PD // 4), row),
            pl.BlockSpec((BLKN, PD // 2), row),
            pl.BlockSpec((BLKN, HD // 4), row),
            pl.BlockSpec((BLKN, HD // 4), row),
        ],
        out_shape=[
            jax.ShapeDtypeStruct((NP_, PD // 4), jnp.int32),
            jax.ShapeDtypeStruct((NP_, PD // 2), jnp.int32),
            jax.ShapeDtypeStruct((NP_, HD // 4), jnp.int32),
            jax.ShapeDtypeStruct((NP_, HD // 4), jnp.int32),
        ],
    )(ofeat, octx, posp, lab, ortp, W_low, b_low, W_high, b_high,
      W_s, b_s, W_o, b_o, W_ws, b_ws, W_wo)


def _make_gather2(slab):
    start, epw, nch, esz = (SL_START[slab], SL_EPW[slab], SL_NCH[slab],
                            SL_E[slab])

    def body(hs, ho, src, dst, g1, g2, idx_s, idx_d, buf1, buf2, sem):
        base = _wid() * epw

        def step(i, _):
            off = base + i * CH
            pltpu.sync_copy(src.at[pl.ds(start + off, CH)], idx_s)
            pltpu.sync_copy(dst.at[pl.ds(start + off, CH)], idx_d)
            c1 = pltpu.async_copy(hs.at[idx_s], buf1, sem)
            c2 = pltpu.async_copy(ho.at[idx_d], buf2, sem)
            c1.wait()
            c2.wait()
            pltpu.sync_copy(buf1, g1.at[pl.ds(off, CH)])
            pltpu.sync_copy(buf2, g2.at[pl.ds(off, CH)])
            return 0

        lax.fori_loop(0, nch, step, 0)

    return functools.partial(
        pl.kernel,
        out_type=[jax.ShapeDtypeStruct((esz, HD // 4), jnp.int32),
                  jax.ShapeDtypeStruct((esz, HD // 4), jnp.int32)],
        mesh=_sc_mesh,
        scratch_types=[
            pltpu.VMEM((CH,), jnp.int32),
            pltpu.VMEM((CH,), jnp.int32),
            pltpu.VMEM((CH, HD // 4), jnp.int32),
            pltpu.VMEM((CH, HD // 4), jnp.int32),
            pltpu.SemaphoreType.DMA,
        ],
    )(body)


_gather2_s = [_make_gather2(0), _make_gather2(1)]


def _edge1_body(u, g1, g2, W_wu, w_hmp, b_hmp, Wr1p, mw_o):
    ub = u[...].astype(jnp.bfloat16)
    wb = W_wu[...].astype(jnp.bfloat16)
    t = jnp.dot(ub, wb, preferred_element_type=_F32)
    h = jnp.tanh(_unpack_f8(g1[...], 1.0 / HS_SCALE)
                 + _unpack_f8(g2[...], 1.0 / HS_SCALE) + t)
    logit = jnp.sum(h * w_hmp[...], axis=1, keepdims=True) + b_hmp[...]
    att = jax.nn.sigmoid(logit)
    r1b = Wr1p[...].astype(jnp.bfloat16)
    mw_o[...] = att * jnp.dot(ub, r1b, preferred_element_type=_F32)


def _edge1(slab, u, g1, g2, W_wu, w_hmp, b_hmp, Wr1p):
    blk0, esz = SL_BLK0[slab], SL_E[slab]
    rowg = lambda i: (i + blk0, 0)
    row = lambda i: (i, 0)
    fixed = lambda i: (0, 0)
    return pl.pallas_call(
        _edge1_body,
        grid=(esz // BLKE,),
        in_specs=[
            pl.BlockSpec((BLKE, PD), rowg),
            pl.BlockSpec((BLKE, HD // 4), row),
            pl.BlockSpec((BLKE, HD // 4), row),
            pl.BlockSpec((PD, HD), fixed),
            pl.BlockSpec((1, HD), fixed),
            pl.BlockSpec((1, 1), fixed),
            pl.BlockSpec((PD, RELW), fixed),
        ],
        out_specs=pl.BlockSpec((BLKE, RELW), row),
        out_shape=jax.ShapeDtypeStruct((esz, RELW), _F32),
    )(u, g1, g2, W_wu, w_hmp, b_hmp, Wr1p)


def _make_scatter(slab):
    start, epw, nch = SL_START[slab], SL_EPW[slab], SL_NCH[slab]
    rps = NP_ // NS

    def body(mw, src, zer, out, idx_v, buf, aggsh):
        cid = lax.axis_index("c")
        sid = lax.axis_index("s")
        wid = sid * NC + cid
        pltpu.sync_copy(zer.at[pl.ds(sid * rps, rps)],
                        aggsh.at[pl.ds(sid * rps, rps)])
        plsc.subcore_barrier()
        base = wid * epw

        def step(i, _):
            off = base + i * CH
            pltpu.sync_copy(src.at[pl.ds(start + off, CH)], idx_v)
            pltpu.sync_copy(mw.at[pl.ds(off, CH)], buf)
            pltpu.sync_copy(buf, aggsh.at[idx_v], add=True)
            return 0

        lax.fori_loop(0, nch, step, 0)
        plsc.subcore_barrier()
        pltpu.sync_copy(aggsh.at[pl.ds(sid * rps, rps)],
                        out.at[pl.ds(cid * NP_ + sid * rps, rps)])

    return functools.partial(
        pl.kernel,
        out_type=jax.ShapeDtypeStruct((2 * NP_, RELW), _F32),
        mesh=_sc_mesh,
        scratch_types=[
            pltpu.VMEM((CH,), jnp.int32),
            pltpu.VMEM((CH, RELW), _F32),
            pltpu.VMEM_SHARED((NP_, RELW), _F32),
        ],
    )(body)


_scatter_s = [_make_scatter(0), _make_scatter(1)]


def _combine_body(pa0, pa1, pb0, pb1, sub, out_o):
    agg = pa0[...] + pa1[...] + pb0[...] + pb1[...]
    out_o[...] = jnp.concatenate([sub[...], _pack_bf16(agg)], axis=1)


def _combine(parts_a, parts_b, sub):
    h0 = lambda i: (i, 0)
    h1 = lambda i: (i + NP_ // BLKN, 0)
    return pl.pallas_call(
        _combine_body,
        grid=(NP_ // BLKN,),
        in_specs=[pl.BlockSpec((BLKN, RELW), h0),
                  pl.BlockSpec((BLKN, RELW), h1),
                  pl.BlockSpec((BLKN, RELW), h0),
                  pl.BlockSpec((BLKN, RELW), h1),
                  pl.BlockSpec((BLKN, PD // 4), h0)],
        out_specs=pl.BlockSpec((BLKN, SRCW), h0),
        out_shape=jax.ShapeDtypeStruct((NP_, SRCW), jnp.int32),
    )(parts_a, parts_a, parts_b, parts_b, sub)


def _make_gather3(slab):
    start, epw, nch, esz = (SL_START[slab], SL_EPW[slab], SL_NCH[slab],
                            SL_E[slab])

    def body(tsrc, obj, src, dst, gs, s2, idx_s, idx_d, b1, b2, sem):
        base = _wid() * epw

        def step(i, _):
            off = base + i * CH
            pltpu.sync_copy(src.at[pl.ds(start + off, CH)], idx_s)
            pltpu.sync_copy(dst.at[pl.ds(start + off, CH)], idx_d)
            c1 = pltpu.async_copy(tsrc.at[idx_s], b1, sem)
            c2 = pltpu.async_copy(obj.at[idx_d], b2, sem)
            c1.wait()
            c2.wait()
            pltpu.sync_copy(b1, gs.at[pl.ds(off, CH)])
            pltpu.sync_copy(b2, s2.at[pl.ds(off, CH)])
            return 0

        lax.fori_loop(0, nch, step, 0)

    return functools.partial(
        pl.kernel,
        out_type=[jax.ShapeDtypeStruct((esz, SRCW), jnp.int32),
                  jax.ShapeDtypeStruct((esz, PD // 2), jnp.int32)],
        mesh=_sc_mesh,
        scratch_types=[
            pltpu.VMEM((CH,), jnp.int32),
            pltpu.VMEM((CH,), jnp.int32),
            pltpu.VMEM((CH, SRCW), jnp.int32),
            pltpu.VMEM((CH, PD // 2), jnp.int32),
            pltpu.SemaphoreType.DMA,
        ],
    )(body)


_gather3_s = [_make_gather3(0), _make_gather3(1)]


def _edge2_body(gs, s2, mw, spt, Wr1p, Wr2p, brel, out_o):
    g = gs[...]
    s1 = _unpack_f8(g[:, :PD // 4], 1.0 / SUB_SCALE)
    ag = _unpack_bf16(g[:, PD // 4:])
    p = s1 * _unpack_bf16(s2[...])
    full = (jnp.dot(p, Wr1p[...], preferred_element_type=_F32)
            + jnp.dot(spt[...], Wr2p[...], preferred_element_type=_F32)
            + ag[:, :RELP] + mw[...][:, :RELP] + brel[...])
    out_o[...] = full[:, :NREL]


def _edge2(slab, gs, s2, mw, spt, Wr1p, Wr2p, brel):
    blk0, esz = SL_BLK0[slab], SL_E[slab]
    rowg = lambda i: (i + blk0, 0)
    row = lambda i: (i, 0)
    fixed = lambda i: (0, 0)
    return pl.pallas_call(
        _edge2_body,
        grid=(esz // BLKE,),
        in_specs=[
            pl.BlockSpec((BLKE, SRCW), row),
            pl.BlockSpec((BLKE, PD // 2), row),
            pl.BlockSpec((BLKE, RELW), row),
            pl.BlockSpec((BLKE, 64), rowg),
            pl.BlockSpec((PD, RELP), fixed),
            pl.BlockSpec((64, RELP), fixed),
            pl.BlockSpec((1, RELP), fixed),
        ],
        out_specs=pl.BlockSpec((BLKE, NREL), row),
        out_shape=jax.ShapeDtypeStruct((esz, NREL), _F32),
    )(gs, s2, mw, spt, Wr1p, Wr2p, brel)


def kernel(obj_feats, obj_ctx, pos_embed, union_feats, spt_feats, pair_idx,
           obj_labels, W_low, b_low, W_high, b_high, W_s, b_s, W_o, b_o,
           W_ws, b_ws, W_wo, W_wu, W_hmp, b_hmp, W_rel, b_rel):
    ind = jnp.arange(1, NUM_OBJ + 1, dtype=_F32)[:, None]
    lin = jnp.linspace(-np.pi, np.pi, ORT_DIMS, dtype=_F32)[None, :]
    t = ind * lin
    ortp = jnp.zeros((256, ORT_DIMS), _F32).at[:NUM_OBJ].set(
        jnp.sin(t) + jnp.cos(t))

    pad_n = NP_ - N
    ofeat = jnp.pad(obj_feats, ((0, pad_n), (0, 0)))
    octx = jnp.pad(obj_ctx, ((0, pad_n), (0, 0)))
    posp = jnp.pad(pos_embed, ((0, pad_n), (0, 3)))
    lab = jnp.pad(obj_labels.astype(jnp.int32), (0, pad_n))[:, None]

    src = pair_idx[:, 0].astype(jnp.int32)
    dst = pair_idx[:, 1].astype(jnp.int32)

    Wr1p = jnp.pad(W_rel[:PD], ((0, 0), (0, RELP - NREL)))
    Wr1w = jnp.pad(W_rel[:PD], ((0, 0), (0, RELW - NREL)))
    Wr2p = jnp.pad(W_rel[PD:], ((0, 0), (0, RELP - NREL)))
    brelp = jnp.pad(b_rel, (0, RELP - NREL))[None, :]
    w_hmp = W_hmp.reshape(1, HD)
    b_hmp2 = b_hmp.reshape(1, 1)
    zer = jnp.zeros((NP_, RELW), _F32)

    sub_p, obj_p, hs_p, ho_p = _node_stage(
        ofeat, octx, posp, lab, ortp, W_low, b_low[None, :], W_high,
        b_high[None, :], W_s, b_s[None, :], W_o, b_o[None, :], W_ws,
        b_ws[None, :], W_wo)

    g1a, g2a = _gather2_s[0](hs_p, ho_p, src, dst)
    g1b, g2b = _gather2_s[1](hs_p, ho_p, src, dst)

    mwa = _edge1(0, union_feats, g1a, g2a, W_wu, w_hmp, b_hmp2, Wr1w)
    mwb = _edge1(1, union_feats, g1b, g2b, W_wu, w_hmp, b_hmp2, Wr1w)

    parts_a = _scatter_s[0](mwa, src, zer)
    parts_b = _scatter_s[1](mwb, src, zer)
    tsrc = _combine(parts_a, parts_b, sub_p)

    gsa, s2a = _gather3_s[0](tsrc, obj_p, src, dst)
    gsb, s2b = _gather3_s[1](tsrc, obj_p, src, dst)

    outa = _edge2(0, gsa, s2a, mwa, spt_feats, Wr1p, Wr2p, brelp)
    outb = _edge2(1, gsb, s2b, mwb, spt_feats, Wr1p, Wr2p, brelp)
    return jnp.concatenate([outa, outb], axis=0)


# speedup vs baseline: 2.9936x; 1.0160x over previous
import functools

import jax
import jax.numpy as jnp
import numpy as np
from jax import lax
from jax.experimental import pallas as pl
from jax.experimental.pallas import tpu as pltpu
from jax.experimental.pallas import tpu_sc as plsc

N = 10000
E = 160000
PD = 256
HD = 512
NUM_OBJ = 151
ORT_DIMS = 200
NREL = 51
RELP = 64
RELW = 128
SRCW = PD // 4 + RELP
DTW = HD // 4 + PD // 2
HS_SCALE = 16.0
SUB_SCALE = 4.0
NP_ = 10240

NC, NS = 2, 16
NW = NC * NS
CH = 200

BLKN = 1024
BLKE = 1600

SL_START = (0, 76800)
SL_E = (76800, 83200)
SL_EPW = (2400, 2600)
SL_NCH = (12, 13)
SL_BLK0 = (0, 48)

_F32 = jnp.float32

_sc_mesh = plsc.VectorSubcoreMesh(
    core_axis_name="c", subcore_axis_name="s", num_cores=NC, num_subcores=NS)


def _wid():
    return lax.axis_index("s") * NC + lax.axis_index("c")


def _pack_bf16(x):
    w = x.shape[1] // 2
    xb = x.astype(jnp.bfloat16)
    lo = lax.bitcast_convert_type(xb[:, :w], jnp.uint16).astype(jnp.uint32)
    hi = lax.bitcast_convert_type(xb[:, w:], jnp.uint16).astype(jnp.uint32)
    return lax.bitcast_convert_type(lo | (hi << 16), jnp.int32)


def _unpack_bf16(p):
    u = lax.bitcast_convert_type(p, jnp.uint32)
    lo = lax.bitcast_convert_type((u & 0xFFFF).astype(jnp.uint16),
                                  jnp.bfloat16).astype(_F32)
    hi = lax.bitcast_convert_type((u >> 16).astype(jnp.uint16),
                                  jnp.bfloat16).astype(_F32)
    return jnp.concatenate([lo, hi], axis=1)


_F8 = jnp.float8_e4m3fn


def _pack_f8(x, scale):
    w = x.shape[1] // 4
    xb = (x * scale).astype(_F8)
    acc = None
    for k in range(4):
        b = lax.bitcast_convert_type(xb[:, k * w:(k + 1) * w],
                                     jnp.uint8).astype(jnp.uint32) << (8 * k)
        acc = b if acc is None else acc | b
    return lax.bitcast_convert_type(acc, jnp.int32)


def _unpack_f8(p, inv_scale):
    u = lax.bitcast_convert_type(p, jnp.uint32)
    parts = []
    for k in range(4):
        b = ((u >> (8 * k)) & 0xFF).astype(jnp.uint8)
        parts.append(
            lax.bitcast_convert_type(b, _F8).astype(_F32) * inv_scale)
    return jnp.concatenate(parts, axis=1)


def _node_body(ofeat, octx, posp, lab, ortp,
               W_low, b_low, W_high, b_high, W_s, b_s, W_o, b_o, W_ws, b_ws,
               W_wo, sub_o, hs_o, dt_o):
    wl = W_low[...]
    label_t = jnp.dot(ortp[...], wl[261:461, :],
                      preferred_element_type=_F32)
    iot = lax.broadcasted_iota(jnp.int32, (BLKN, 256), 1)
    oh = (iot == lab[...]).astype(_F32)
    rep = (jnp.dot(ofeat[...], wl[:256, :], preferred_element_type=_F32)
           + jnp.dot(posp[...], wl[256:264, :], preferred_element_type=_F32)
           + jnp.dot(oh, label_t, preferred_element_type=_F32)
           + b_low[...]
           + jnp.dot(octx[...], W_high[...], preferred_element_type=_F32)
           + b_high[...])
    sub = jnp.dot(rep, W_s[...], preferred_element_type=_F32) + b_s[...]
    obj = jnp.dot(rep, W_o[...], preferred_element_type=_F32) + b_o[...]
    sub_o[...] = _pack_f8(sub, SUB_SCALE)
    hs_o[...] = _pack_f8(
        jnp.dot(sub, W_ws[...], preferred_element_type=_F32), HS_SCALE)
    ho = _pack_f8(
        jnp.dot(obj, W_wo[...], preferred_element_type=_F32) + b_ws[...],
        HS_SCALE)
    dt_o[...] = jnp.concatenate([ho, _pack_bf16(obj)], axis=1)


def _node_stage(ofeat, octx, posp, lab, ortp, W_low, b_low, W_high, b_high,
                W_s, b_s, W_o, b_o, W_ws, b_ws, W_wo):
    row = lambda i: (i, 0)
    fixed = lambda i: (0, 0)
    return pl.pallas_call(
        _node_body,
        grid=(NP_ // BLKN,),
        in_specs=[
            pl.BlockSpec((BLKN, PD), row),
            pl.BlockSpec((BLKN, HD), row),
            pl.BlockSpec((BLKN, 8), row),
            pl.BlockSpec((BLKN, 1), row),
            pl.BlockSpec((256, ORT_DIMS), fixed),
            pl.BlockSpec((461, PD), fixed),
            pl.BlockSpec((1, PD), fixed),
            pl.BlockSpec((HD, PD), fixed),
            pl.BlockSpec((1, PD), fixed),
            pl.BlockSpec((PD, PD), fixed),
            pl.BlockSpec((1, PD), fixed),
            pl.BlockSpec((PD, PD), fixed),
            pl.BlockSpec((1, PD), fixed),
            pl.BlockSpec((PD, HD), fixed),
            pl.BlockSpec((1, HD), fixed),
            pl.BlockSpec((PD, HD), fixed),
        ],
        out_specs=[
            pl.BlockSpec((BLKN, PD // 4), row),
            pl.BlockSpec((BLKN, HD // 4), row),
            pl.BlockSpec((BLKN, DTW), row),
        ],
        out_shape=[
            jax.ShapeDtypeStruct((NP_, PD // 4), jnp.int32),
            jax.ShapeDtypeStruct((NP_, HD // 4), jnp.int32),
            jax.ShapeDtypeStruct((NP_, DTW), jnp.int32),
        ],
    )(ofeat, octx, posp, lab, ortp, W_low, b_low, W_high, b_high,
      W_s, b_s, W_o, b_o, W_ws, b_ws, W_wo)


def _make_gather2(slab):
    start, epw, nch, esz = (SL_START[slab], SL_EPW[slab], SL_NCH[slab],
                            SL_E[slab])

    def body(hs, dt, src, dst, g1, g2, idx_s, idx_d, buf1, buf2, sem):
        base = _wid() * epw

        def step(i, _):
            off = base + i * CH
            pltpu.sync_copy(src.at[pl.ds(start + off, CH)], idx_s)
            pltpu.sync_copy(dst.at[pl.ds(start + off, CH)], idx_d)
            c1 = pltpu.async_copy(hs.at[idx_s], buf1, sem)
            c2 = pltpu.async_copy(dt.at[idx_d], buf2, sem)
            c1.wait()
            c2.wait()
            pltpu.sync_copy(buf1, g1.at[pl.ds(off, CH)])
            pltpu.sync_copy(buf2, g2.at[pl.ds(off, CH)])
            return 0

        lax.fori_loop(0, nch, step, 0)

    return functools.partial(
        pl.kernel,
        out_type=[jax.ShapeDtypeStruct((esz, HD // 4), jnp.int32),
                  jax.ShapeDtypeStruct((esz, DTW), jnp.int32)],
        mesh=_sc_mesh,
        scratch_types=[
            pltpu.VMEM((CH,), jnp.int32),
            pltpu.VMEM((CH,), jnp.int32),
            pltpu.VMEM((CH, HD // 4), jnp.int32),
            pltpu.VMEM((CH, DTW), jnp.int32),
            pltpu.SemaphoreType.DMA,
        ],
    )(body)


_gather2_s = [_make_gather2(0), _make_gather2(1)]


def _edge1_body(u, g1, g2, W_wu, w_hmp, b_hmp, Wr1p, mw_o):
    ub = u[...].astype(jnp.bfloat16)
    wb = W_wu[...].astype(jnp.bfloat16)
    t = jnp.dot(ub, wb, preferred_element_type=_F32)
    h = jnp.tanh(_unpack_f8(g1[...], 1.0 / HS_SCALE)
                 + _unpack_f8(g2[...], 1.0 / HS_SCALE) + t)
    logit = jnp.sum(h * w_hmp[...], axis=1, keepdims=True) + b_hmp[...]
    att = jax.nn.sigmoid(logit)
    r1b = Wr1p[...].astype(jnp.bfloat16)
    mw_o[...] = att * jnp.dot(ub, r1b, preferred_element_type=_F32)


def _edge1(slab, u, g1, g2, W_wu, w_hmp, b_hmp, Wr1p):
    blk0, esz = SL_BLK0[slab], SL_E[slab]
    rowg = lambda i: (i + blk0, 0)
    row = lambda i: (i, 0)
    fixed = lambda i: (0, 0)
    return pl.pallas_call(
        _edge1_body,
        grid=(esz // BLKE,),
        in_specs=[
            pl.BlockSpec((BLKE, PD), rowg),
            pl.BlockSpec((BLKE, HD // 4), row),
            pl.BlockSpec((BLKE, HD // 4), row),
            pl.BlockSpec((PD, HD), fixed),
            pl.BlockSpec((1, HD), fixed),
            pl.BlockSpec((1, 1), fixed),
            pl.BlockSpec((PD, RELW), fixed),
        ],
        out_specs=pl.BlockSpec((BLKE, RELW), row),
        out_shape=jax.ShapeDtypeStruct((esz, RELW), _F32),
    )(u, g1, g2, W_wu, w_hmp, b_hmp, Wr1p)


def _make_scatter(slab):
    start, epw, nch = SL_START[slab], SL_EPW[slab], SL_NCH[slab]
    rps = NP_ // NS

    def body(mw, src, zer, out, idx_v, buf, aggsh):
        cid = lax.axis_index("c")
        sid = lax.axis_index("s")
        wid = sid * NC + cid
        pltpu.sync_copy(zer.at[pl.ds(sid * rps, rps)],
                        aggsh.at[pl.ds(sid * rps, rps)])
        plsc.subcore_barrier()
        base = wid * epw

        def step(i, _):
            off = base + i * CH
            pltpu.sync_copy(src.at[pl.ds(start + off, CH)], idx_v)
            pltpu.sync_copy(mw.at[pl.ds(off, CH)], buf)
            pltpu.sync_copy(buf, aggsh.at[idx_v], add=True)
            return 0

        lax.fori_loop(0, nch, step, 0)
        plsc.subcore_barrier()
        pltpu.sync_copy(aggsh.at[pl.ds(sid * rps, rps)],
                        out.at[pl.ds(cid * NP_ + sid * rps, rps)])

    return functools.partial(
        pl.kernel,
        out_type=jax.ShapeDtypeStruct((2 * NP_, RELW), _F32),
        mesh=_sc_mesh,
        scratch_types=[
            pltpu.VMEM((CH,), jnp.int32),
            pltpu.VMEM((CH, RELW), _F32),
            pltpu.VMEM_SHARED((NP_, RELW), _F32),
        ],
    )(body)


_scatter_s = [_make_scatter(0), _make_scatter(1)]


def _combine_body(pa0, pa1, pb0, pb1, sub, out_o):
    agg = pa0[...] + pa1[...] + pb0[...] + pb1[...]
    out_o[...] = jnp.concatenate([sub[...], _pack_bf16(agg)], axis=1)


def _combine(parts_a, parts_b, sub):
    h0 = lambda i: (i, 0)
    h1 = lambda i: (i + NP_ // BLKN, 0)
    return pl.pallas_call(
        _combine_body,
        grid=(NP_ // BLKN,),
        in_specs=[pl.BlockSpec((BLKN, RELW), h0),
                  pl.BlockSpec((BLKN, RELW), h1),
                  pl.BlockSpec((BLKN, RELW), h0),
                  pl.BlockSpec((BLKN, RELW), h1),
                  pl.BlockSpec((BLKN, PD // 4), h0)],
        out_specs=pl.BlockSpec((BLKN, SRCW), h0),
        out_shape=jax.ShapeDtypeStruct((NP_, SRCW), jnp.int32),
    )(parts_a, parts_a, parts_b, parts_b, sub)


def _make_gather3(slab):
    start, epw, nch, esz = (SL_START[slab], SL_EPW[slab], SL_NCH[slab],
                            SL_E[slab])

    def body(tsrc, src, gs, idx_s, b1, sem):
        base = _wid() * epw

        def step(i, _):
            off = base + i * CH
            pltpu.sync_copy(src.at[pl.ds(start + off, CH)], idx_s)
            c1 = pltpu.async_copy(tsrc.at[idx_s], b1, sem)
            c1.wait()
            pltpu.sync_copy(b1, gs.at[pl.ds(off, CH)])
            return 0

        lax.fori_loop(0, nch, step, 0)

    return functools.partial(
        pl.kernel,
        out_type=jax.ShapeDtypeStruct((esz, SRCW), jnp.int32),
        mesh=_sc_mesh,
        scratch_types=[
            pltpu.VMEM((CH,), jnp.int32),
            pltpu.VMEM((CH, SRCW), jnp.int32),
            pltpu.SemaphoreType.DMA,
        ],
    )(body)


_gather3_s = [_make_gather3(0), _make_gather3(1)]


def _edge2_body(gs, g2, mw, spt, Wr1p, Wr2p, brel, out_o):
    g = gs[...]
    s1 = _unpack_f8(g[:, :PD // 4], 1.0 / SUB_SCALE)
    ag = _unpack_bf16(g[:, PD // 4:])
    p = s1 * _unpack_bf16(g2[...])
    full = (jnp.dot(p, Wr1p[...], preferred_element_type=_F32)
            + jnp.dot(spt[...], Wr2p[...], preferred_element_type=_F32)
            + ag[:, :RELP] + mw[...][:, :RELP] + brel[...])
    out_o[...] = full[:, :NREL]


def _edge2(slab, gs, g2, mw, spt, Wr1p, Wr2p, brel):
    blk0, esz = SL_BLK0[slab], SL_E[slab]
    rowg = lambda i: (i + blk0, 0)
    row = lambda i: (i, 0)
    row1 = lambda i: (i, 1)
    fixed = lambda i: (0, 0)
    return pl.pallas_call(
        _edge2_body,
        grid=(esz // BLKE,),
        in_specs=[
            pl.BlockSpec((BLKE, SRCW), row),
            pl.BlockSpec((BLKE, PD // 2), row1),
            pl.BlockSpec((BLKE, RELW), row),
            pl.BlockSpec((BLKE, 64), rowg),
            pl.BlockSpec((PD, RELP), fixed),
            pl.BlockSpec((64, RELP), fixed),
            pl.BlockSpec((1, RELP), fixed),
        ],
        out_specs=pl.BlockSpec((BLKE, NREL), row),
        out_shape=jax.ShapeDtypeStruct((esz, NREL), _F32),
    )(gs, g2, mw, spt, Wr1p, Wr2p, brel)


def kernel(obj_feats, obj_ctx, pos_embed, union_feats, spt_feats, pair_idx,
           obj_labels, W_low, b_low, W_high, b_high, W_s, b_s, W_o, b_o,
           W_ws, b_ws, W_wo, W_wu, W_hmp, b_hmp, W_rel, b_rel):
    ind = jnp.arange(1, NUM_OBJ + 1, dtype=_F32)[:, None]
    lin = jnp.linspace(-np.pi, np.pi, ORT_DIMS, dtype=_F32)[None, :]
    t = ind * lin
    ortp = jnp.zeros((256, ORT_DIMS), _F32).at[:NUM_OBJ].set(
        jnp.sin(t) + jnp.cos(t))

    pad_n = NP_ - N
    ofeat = jnp.pad(obj_feats, ((0, pad_n), (0, 0)))
    octx = jnp.pad(obj_ctx, ((0, pad_n), (0, 0)))
    posp = jnp.pad(pos_embed, ((0, pad_n), (0, 3)))
    lab = jnp.pad(obj_labels.astype(jnp.int32), (0, pad_n))[:, None]

    src = pair_idx[:, 0].astype(jnp.int32)
    dst = pair_idx[:, 1].astype(jnp.int32)

    Wr1p = jnp.pad(W_rel[:PD], ((0, 0), (0, RELP - NREL)))
    Wr1w = jnp.pad(W_rel[:PD], ((0, 0), (0, RELW - NREL)))
    Wr2p = jnp.pad(W_rel[PD:], ((0, 0), (0, RELP - NREL)))
    brelp = jnp.pad(b_rel, (0, RELP - NREL))[None, :]
    w_hmp = W_hmp.reshape(1, HD)
    b_hmp2 = b_hmp.reshape(1, 1)
    zer = jnp.zeros((NP_, RELW), _F32)

    sub_p, hs_p, dt_p = _node_stage(
        ofeat, octx, posp, lab, ortp, W_low, b_low[None, :], W_high,
        b_high[None, :], W_s, b_s[None, :], W_o, b_o[None, :], W_ws,
        b_ws[None, :], W_wo)

    g1a, g2a = _gather2_s[0](hs_p, dt_p, src, dst)
    g1b, g2b = _gather2_s[1](hs_p, dt_p, src, dst)

    mwa = _edge1(0, union_feats, g1a, g2a, W_wu, w_hmp, b_hmp2, Wr1w)
    mwb = _edge1(1, union_feats, g1b, g2b, W_wu, w_hmp, b_hmp2, Wr1w)

    parts_a = _scatter_s[0](mwa, src, zer)
    parts_b = _scatter_s[1](mwb, src, zer)
    tsrc = _combine(parts_a, parts_b, sub_p)

    gsa = _gather3_s[0](tsrc, src)
    gsb = _gather3_s[1](tsrc, src)

    outa = _edge2(0, gsa, g2a, mwa, spt_feats, Wr1p, Wr2p, brelp)
    outb = _edge2(1, gsb, g2b, mwb, spt_feats, Wr1p, Wr2p, brelp)
    return jnp.concatenate([outa, outb], axis=0)
